# Initial kernel scaffold; baseline (speedup 1.0000x reference)
import functools

import jax
import jax.numpy as jnp
from jax import lax
from jax.experimental import pallas as pl
from jax.experimental.pallas import tpu as pltpu
from jax.experimental.pallas import tpu_sc as plsc

LANES = 16
TILES = 32


def _sc_mesh():
    return plsc.VectorSubcoreMesh(
        core_axis_name="c", subcore_axis_name="s", num_cores=2, num_subcores=16
    )


def _pick_edge_chunk(ep: int, word_budget: int) -> int:
    for nch in range(2, 4096, 2):
        if ep % nch == 0:
            ce = ep // nch
            if ce % LANES == 0 and ce <= word_budget:
                return ce
    raise ValueError(f"no edge chunking for {ep}")


def _make_hist(n: int, np_: int, ep: int):
    et = ep // TILES

    @functools.partial(
        pl.kernel,
        out_type=jax.ShapeDtypeStruct((TILES, n), jnp.int32),
        mesh=_sc_mesh(),
        scratch_types=[
            pltpu.VMEM((np_,), jnp.int32),
            pltpu.VMEM((et,), jnp.int32),
        ],
    )
    def hist(pe_hbm, out_hbm, hist_v, ebuf):
        wid = lax.axis_index("s") * 2 + lax.axis_index("c")
        zeros = jnp.zeros((LANES,), jnp.int32)
        ones = jnp.ones((LANES,), jnp.int32)

        @pl.loop(0, np_, step=LANES)
        def _(i):
            hist_v[pl.ds(i, LANES)] = zeros

        pltpu.sync_copy(pe_hbm.at[pl.ds(wid * et, et)], ebuf)

        @pl.loop(0, et, step=LANES)
        def _(i):
            pe = ebuf[pl.ds(i, LANES)]
            d = (pe >> 16) & 0xFFFF
            plsc.addupdate_scatter(hist_v, [d], ones)

        pltpu.sync_copy(hist_v.at[pl.ds(0, n)], out_hbm.at[wid])

    return hist


def _make_agg(f_dim: int, n: int, np_: int, ep: int, ce: int):
    nch = ep // ce
    npass = (f_dim + TILES - 1) // TILES

    @functools.partial(
        pl.kernel,
        out_type=jax.ShapeDtypeStruct((f_dim, n), jnp.float32),
        mesh=_sc_mesh(),
        scratch_types=[
            pltpu.VMEM((np_,), jnp.float32),
            pltpu.VMEM((np_,), jnp.float32),
            pltpu.VMEM((ce,), jnp.int32),
            pltpu.VMEM((ce,), jnp.int32),
            pltpu.SemaphoreType.DMA,
            pltpu.SemaphoreType.DMA,
        ],
    )
    def agg(y_hbm, pe_hbm, out_hbm, col_v, acc_v, eb0, eb1, sem0, sem1):
        wid = lax.axis_index("s") * 2 + lax.axis_index("c")
        zeros = jnp.zeros((LANES,), jnp.float32)

        def consume(ebuf):
            @pl.loop(0, ce, step=LANES)
            def _(i):
                pe = ebuf[pl.ds(i, LANES)]
                s = pe & 0xFFFF
                d = (pe >> 16) & 0xFFFF
                v = plsc.load_gather(col_v, [s])
                plsc.addupdate_scatter(acc_v, [d], v)

        for p in range(npass):
            feat = wid + TILES * p

            @pl.when(feat < f_dim)
            def _():
                pltpu.sync_copy(y_hbm.at[feat], col_v.at[pl.ds(0, n)])
                col_v[pl.ds(n, np_ - n)] = jnp.zeros((np_ - n,), jnp.float32)

                @pl.loop(0, np_, step=LANES)
                def _(i):
                    acc_v[pl.ds(i, LANES)] = zeros

                pltpu.async_copy(pe_hbm.at[pl.ds(0, ce)], eb0, sem0)

                @pl.loop(0, nch, step=2)
                def _(c):
                    pltpu.async_copy(pe_hbm.at[pl.ds((c + 1) * ce, ce)], eb1, sem1)
                    pltpu.make_async_copy(
                        pe_hbm.at[pl.ds(c * ce, ce)], eb0, sem0
                    ).wait()
                    consume(eb0)

                    @pl.when(c + 2 < nch)
                    def _():
                        pltpu.async_copy(
                            pe_hbm.at[pl.ds((c + 2) * ce, ce)], eb0, sem0
                        )

                    pltpu.make_async_copy(
                        pe_hbm.at[pl.ds((c + 1) * ce, ce)], eb1, sem1
                    ).wait()
                    consume(eb1)

                pltpu.sync_copy(acc_v.at[pl.ds(0, n)], out_hbm.at[feat])

    return agg


_HIGHEST = jax.lax.Precision.HIGHEST


def _mm_t(w, x, bn):
    k_dim, h_dim = w.shape
    n = x.shape[0]

    def body(w_ref, x_ref, o_ref):
        o_ref[...] = lax.dot_general(
            w_ref[...],
            x_ref[...],
            (((0,), (1,)), ((), ())),
            precision=_HIGHEST,
            preferred_element_type=jnp.float32,
        )

    return pl.pallas_call(
        body,
        grid=(n // bn,),
        in_specs=[
            pl.BlockSpec((k_dim, h_dim), lambda i: (0, 0)),
            pl.BlockSpec((bn, k_dim), lambda i: (i, 0)),
        ],
        out_specs=pl.BlockSpec((h_dim, bn), lambda i: (0, i)),
        out_shape=jax.ShapeDtypeStruct((h_dim, n), jnp.float32),
    )(w, x)


def _scale(hist, xw_t, bn):
    h_dim, n = xw_t.shape

    def body(hist_ref, xw_ref, y_ref, dinv_ref):
        deg = jnp.sum(hist_ref[...].astype(jnp.float32), axis=0, keepdims=True)
        dinv = lax.rsqrt(deg + 1.0)
        dinv_ref[...] = dinv
        y_ref[...] = xw_ref[...] * dinv

    return pl.pallas_call(
        body,
        grid=(n // bn,),
        in_specs=[
            pl.BlockSpec((TILES, bn), lambda i: (0, i)),
            pl.BlockSpec((h_dim, bn), lambda i: (0, i)),
        ],
        out_specs=[
            pl.BlockSpec((h_dim, bn), lambda i: (0, i)),
            pl.BlockSpec((1, bn), lambda i: (0, i)),
        ],
        out_shape=[
            jax.ShapeDtypeStruct((h_dim, n), jnp.float32),
            jax.ShapeDtypeStruct((1, n), jnp.float32),
        ],
    )(hist, xw_t)


def _layer2(agg1_t, y1_t, dinv, b1_col, w2, bn):
    h_dim, n = y1_t.shape
    c_dim = w2.shape[1]

    def body(agg_ref, y1_ref, dinv_ref, b1_ref, w2_ref, y2_ref):
        dinv_blk = dinv_ref[...]
        h = jnp.maximum(
            dinv_blk * (agg_ref[...] + y1_ref[...]) + b1_ref[...], 0.0
        )
        y2 = lax.dot_general(
            w2_ref[...],
            h,
            (((0,), (0,)), ((), ())),
            precision=_HIGHEST,
            preferred_element_type=jnp.float32,
        )
        y2_ref[...] = y2 * dinv_blk

    return pl.pallas_call(
        body,
        grid=(n // bn,),
        in_specs=[
            pl.BlockSpec((h_dim, bn), lambda i: (0, i)),
            pl.BlockSpec((h_dim, bn), lambda i: (0, i)),
            pl.BlockSpec((1, bn), lambda i: (0, i)),
            pl.BlockSpec((h_dim, 1), lambda i: (0, 0)),
            pl.BlockSpec((h_dim, c_dim), lambda i: (0, 0)),
        ],
        out_specs=pl.BlockSpec((c_dim, bn), lambda i: (0, i)),
        out_shape=jax.ShapeDtypeStruct((c_dim, n), jnp.float32),
    )(agg1_t, y1_t, dinv, b1_col, w2)


def _final(agg2_t, y2_t, dinv, b2_col, bn):
    c_dim, n = y2_t.shape

    def body(agg_ref, y2_ref, dinv_ref, b2_ref, o_ref):
        o_ref[...] = (
            dinv_ref[...] * (agg_ref[...] + y2_ref[...]) + b2_ref[...]
        )

    return pl.pallas_call(
        body,
        grid=(n // bn,),
        in_specs=[
            pl.BlockSpec((c_dim, bn), lambda i: (0, i)),
            pl.BlockSpec((c_dim, bn), lambda i: (0, i)),
            pl.BlockSpec((1, bn), lambda i: (0, i)),
            pl.BlockSpec((c_dim, 1), lambda i: (0, 0)),
        ],
        out_specs=pl.BlockSpec((c_dim, bn), lambda i: (0, i)),
        out_shape=jax.ShapeDtypeStruct((c_dim, n), jnp.float32),
    )(agg2_t, y2_t, dinv, b2_col)


def kernel(x, edge_index, W1, b1, W2, b2):
    n, f_in = x.shape
    h_dim = W1.shape[1]
    c_dim = W2.shape[1]
    e = edge_index.shape[1]

    src = edge_index[0].astype(jnp.int32)
    dst = edge_index[1].astype(jnp.int32)
    pe = jnp.bitwise_or(src, jnp.left_shift(dst, 16))

    group = TILES * LANES
    ep = ((e + group - 1) // group) * group
    if ep != e:
        sent = jnp.full((ep - e,), n, jnp.int32)
        pe = jnp.concatenate([pe, jnp.bitwise_or(sent, jnp.left_shift(sent, 16))])
    np_ = n + LANES

    word_budget = (131000 - 2 * np_) // 2
    ce = _pick_edge_chunk(ep, word_budget)

    bn = 2000 if n % 2000 == 0 else max(
        b for b in (1000, 500, 250, 200, 100, 50, 40, 25, 8) if n % b == 0
    )

    hist = _make_hist(n, np_, ep)(pe)
    xw1_t = _mm_t(W1, x, bn)
    y1_t, dinv = _scale(hist, xw1_t, bn)
    agg1_t = _make_agg(h_dim, n, np_, ep, ce)(y1_t, pe)
    y2_t = _layer2(agg1_t, y1_t, dinv, b1.reshape(h_dim, 1), W2, bn)
    agg2_t = _make_agg(c_dim, n, np_, ep, ce)(y2_t, pe)
    out_t = _final(agg2_t, y2_t, dinv, b2.reshape(c_dim, 1), bn)
    return out_t.T


# speedup vs baseline: 7.8995x; 7.8995x over previous
import dataclasses
import functools

import jax
import jax.numpy as jnp
from jax import lax
from jax.experimental import pallas as pl
from jax.experimental.pallas import tpu as pltpu
from jax.experimental.pallas import tpu_sc as plsc

LANES = 16
TILES = 32


def _sc_mesh():
    return plsc.VectorSubcoreMesh(
        core_axis_name="c", subcore_axis_name="s", num_cores=2, num_subcores=16
    )


def _sc_params():
    cp = pltpu.CompilerParams()
    if "needs_layout_passes" in pltpu.CompilerParams.__dataclass_fields__:
        cp = dataclasses.replace(cp, needs_layout_passes=False)
    return cp


def _pick_edge_chunk(ep: int, word_budget: int) -> int:
    for nch in range(2, 4096, 2):
        if ep % nch == 0:
            ce = ep // nch
            if ce % LANES == 0 and ce <= word_budget:
                return ce
    raise ValueError(f"no edge chunking for {ep}")


def _make_hist(n: int, np_: int, ep: int):
    et = ep // TILES

    @functools.partial(
        pl.kernel,
        out_type=jax.ShapeDtypeStruct((TILES * n,), jnp.int32),
        mesh=_sc_mesh(),
        scratch_types=[
            pltpu.VMEM((np_,), jnp.int32),
            pltpu.VMEM((et,), jnp.int32),
        ],
        compiler_params=_sc_params(),
    )
    def hist(pe_hbm, out_hbm, hist_v, ebuf):
        wid = lax.axis_index("s") * 2 + lax.axis_index("c")
        zeros = jnp.zeros((LANES,), jnp.int32)
        ones = jnp.ones((LANES,), jnp.int32)

        @pl.loop(0, np_, step=LANES)
        def _(i):
            hist_v[pl.ds(i, LANES)] = zeros

        pltpu.sync_copy(pe_hbm.at[pl.ds(wid * et, et)], ebuf)

        @pl.loop(0, et, step=LANES)
        def _(i):
            pe = ebuf[pl.ds(i, LANES)]
            d = (pe >> 16) & 0xFFFF
            plsc.addupdate_scatter(hist_v, [d], ones)

        pltpu.sync_copy(hist_v.at[pl.ds(0, n)], out_hbm.at[pl.ds(wid * n, n)])

    return hist


def _make_agg(f_dim: int, n: int, np_: int, ep: int, ce: int):
    nch = ep // ce
    npass = (f_dim + TILES - 1) // TILES

    @functools.partial(
        pl.kernel,
        out_type=jax.ShapeDtypeStruct((f_dim * n,), jnp.float32),
        mesh=_sc_mesh(),
        scratch_types=[
            pltpu.VMEM((np_,), jnp.float32),
            pltpu.VMEM((np_,), jnp.float32),
            pltpu.VMEM((ce,), jnp.int32),
            pltpu.VMEM((ce,), jnp.int32),
            pltpu.SemaphoreType.DMA,
            pltpu.SemaphoreType.DMA,
        ],
        compiler_params=_sc_params(),
    )
    def agg(y_hbm, pe_hbm, out_hbm, col_v, acc_v, eb0, eb1, sem0, sem1):
        wid = lax.axis_index("s") * 2 + lax.axis_index("c")
        zeros = jnp.zeros((LANES,), jnp.float32)

        def consume(ebuf):
            @pl.loop(0, ce, step=LANES)
            def _(i):
                pe = ebuf[pl.ds(i, LANES)]
                s = pe & 0xFFFF
                d = (pe >> 16) & 0xFFFF
                v = plsc.load_gather(col_v, [s])
                plsc.addupdate_scatter(acc_v, [d], v)

        for p in range(npass):
            feat = wid + TILES * p

            @pl.when(feat < f_dim)
            def _():
                pltpu.sync_copy(y_hbm.at[pl.ds(feat * n, n)], col_v.at[pl.ds(0, n)])
                col_v[pl.ds(n, np_ - n)] = jnp.zeros((np_ - n,), jnp.float32)

                @pl.loop(0, np_, step=LANES)
                def _(i):
                    acc_v[pl.ds(i, LANES)] = zeros

                pltpu.async_copy(pe_hbm.at[pl.ds(0, ce)], eb0, sem0)

                @pl.loop(0, nch, step=2)
                def _(c):
                    pltpu.async_copy(pe_hbm.at[pl.ds((c + 1) * ce, ce)], eb1, sem1)
                    pltpu.make_async_copy(
                        pe_hbm.at[pl.ds(c * ce, ce)], eb0, sem0
                    ).wait()
                    consume(eb0)

                    @pl.when(c + 2 < nch)
                    def _():
                        pltpu.async_copy(
                            pe_hbm.at[pl.ds((c + 2) * ce, ce)], eb0, sem0
                        )

                    pltpu.make_async_copy(
                        pe_hbm.at[pl.ds((c + 1) * ce, ce)], eb1, sem1
                    ).wait()
                    consume(eb1)

                pltpu.sync_copy(acc_v.at[pl.ds(0, n)], out_hbm.at[pl.ds(feat * n, n)])

    return agg


_HIGHEST = jax.lax.Precision.HIGHEST


def _mm1(x, w1, bn):
    n, k_dim = x.shape
    h_dim = w1.shape[1]

    def body(x_ref, w_ref, o_ref):
        o_ref[...] = lax.dot_general(
            x_ref[...],
            w_ref[...],
            (((1,), (0,)), ((), ())),
            precision=_HIGHEST,
            preferred_element_type=jnp.float32,
        )

    return pl.pallas_call(
        body,
        grid=(n // bn,),
        in_specs=[
            pl.BlockSpec((bn, k_dim), lambda i: (i, 0)),
            pl.BlockSpec((k_dim, h_dim), lambda i: (0, 0)),
        ],
        out_specs=pl.BlockSpec((bn, h_dim), lambda i: (i, 0)),
        out_shape=jax.ShapeDtypeStruct((n, h_dim), jnp.float32),
    )(x, w1)


def _scale(hist_t, xw1, bn):
    n, h_dim = xw1.shape

    def body(hist_ref, xw_ref, y_ref, dinv_ref):
        deg = jnp.sum(hist_ref[...].astype(jnp.float32), axis=1, keepdims=True)
        dinv = lax.rsqrt(deg + 1.0)
        dinv_ref[...] = dinv
        y_ref[...] = xw_ref[...] * dinv

    return pl.pallas_call(
        body,
        grid=(n // bn,),
        in_specs=[
            pl.BlockSpec((bn, TILES), lambda i: (i, 0)),
            pl.BlockSpec((bn, h_dim), lambda i: (i, 0)),
        ],
        out_specs=[
            pl.BlockSpec((bn, h_dim), lambda i: (i, 0)),
            pl.BlockSpec((bn, 1), lambda i: (i, 0)),
        ],
        out_shape=[
            jax.ShapeDtypeStruct((n, h_dim), jnp.float32),
            jax.ShapeDtypeStruct((n, 1), jnp.float32),
        ],
    )(hist_t, xw1)


def _layer2(agg1, y1, dinv, b1_row, w2, bn):
    n, h_dim = y1.shape
    c_dim = w2.shape[1]

    def body(agg_ref, y1_ref, dinv_ref, b1_ref, w2_ref, y2_ref):
        dinv_blk = dinv_ref[...]
        h = jnp.maximum(
            dinv_blk * (agg_ref[...] + y1_ref[...]) + b1_ref[...], 0.0
        )
        y2 = lax.dot_general(
            h,
            w2_ref[...],
            (((1,), (0,)), ((), ())),
            precision=_HIGHEST,
            preferred_element_type=jnp.float32,
        )
        y2_ref[...] = y2 * dinv_blk

    return pl.pallas_call(
        body,
        grid=(n // bn,),
        in_specs=[
            pl.BlockSpec((bn, h_dim), lambda i: (i, 0)),
            pl.BlockSpec((bn, h_dim), lambda i: (i, 0)),
            pl.BlockSpec((bn, 1), lambda i: (i, 0)),
            pl.BlockSpec((1, h_dim), lambda i: (0, 0)),
            pl.BlockSpec((h_dim, c_dim), lambda i: (0, 0)),
        ],
        out_specs=pl.BlockSpec((bn, c_dim), lambda i: (i, 0)),
        out_shape=jax.ShapeDtypeStruct((n, c_dim), jnp.float32),
    )(agg1, y1, dinv, b1_row, w2)


def _final(agg2, y2, dinv, b2_row, bn):
    n, c_dim = y2.shape

    def body(agg_ref, y2_ref, dinv_ref, b2_ref, o_ref):
        o_ref[...] = (
            dinv_ref[...] * (agg_ref[...] + y2_ref[...]) + b2_ref[...]
        )

    return pl.pallas_call(
        body,
        grid=(n // bn,),
        in_specs=[
            pl.BlockSpec((bn, c_dim), lambda i: (i, 0)),
            pl.BlockSpec((bn, c_dim), lambda i: (i, 0)),
            pl.BlockSpec((bn, 1), lambda i: (i, 0)),
            pl.BlockSpec((1, c_dim), lambda i: (0, 0)),
        ],
        out_specs=pl.BlockSpec((bn, c_dim), lambda i: (i, 0)),
        out_shape=jax.ShapeDtypeStruct((n, c_dim), jnp.float32),
    )(agg2, y2, dinv, b2_row)


def kernel(x, edge_index, W1, b1, W2, b2):
    n, f_in = x.shape
    h_dim = W1.shape[1]
    c_dim = W2.shape[1]
    e = edge_index.shape[1]

    src = edge_index[0].astype(jnp.int32)
    dst = edge_index[1].astype(jnp.int32)
    pe = jnp.bitwise_or(src, jnp.left_shift(dst, 16))

    group = TILES * LANES
    ep = ((e + group - 1) // group) * group
    if ep != e:
        sent = jnp.full((ep - e,), n, jnp.int32)
        pe = jnp.concatenate([pe, jnp.bitwise_or(sent, jnp.left_shift(sent, 16))])
    np_ = n + LANES

    word_budget = (131000 - 2 * np_) // 2
    ce = _pick_edge_chunk(ep, word_budget)

    bn = 2000 if n % 2000 == 0 else max(
        b for b in (1000, 500, 250, 200, 100, 50, 40, 25, 8) if n % b == 0
    )

    hist = _make_hist(n, np_, ep)(pe).reshape(TILES, n)
    xw1 = _mm1(x, W1, bn)
    y1, dinv = _scale(hist.T, xw1, bn)
    agg1_t = _make_agg(h_dim, n, np_, ep, ce)(y1.T.reshape(-1), pe)
    agg1 = agg1_t.reshape(h_dim, n).T
    y2 = _layer2(agg1, y1, dinv, b1.reshape(1, h_dim), W2, bn)
    agg2_t = _make_agg(c_dim, n, np_, ep, ce)(y2.T.reshape(-1), pe)
    agg2 = agg2_t.reshape(c_dim, n).T
    return _final(agg2, y2, dinv, b2.reshape(1, c_dim), bn)


# speedup vs baseline: 8.1787x; 1.0353x over previous
import dataclasses
import functools

import jax
import jax.numpy as jnp
from jax import lax
from jax.experimental import pallas as pl
from jax.experimental.pallas import tpu as pltpu
from jax.experimental.pallas import tpu_sc as plsc

LANES = 16
TILES = 32


def _sc_mesh():
    return plsc.VectorSubcoreMesh(
        core_axis_name="c", subcore_axis_name="s", num_cores=2, num_subcores=16
    )


def _sc_params():
    cp = pltpu.CompilerParams()
    if "needs_layout_passes" in pltpu.CompilerParams.__dataclass_fields__:
        cp = dataclasses.replace(cp, needs_layout_passes=False)
    return cp


UNROLL = 8
STEP = UNROLL * LANES
CE = 8192


def _make_hist(n: int, np_: int, ep: int):
    et = ep // TILES

    @functools.partial(
        pl.kernel,
        out_type=jax.ShapeDtypeStruct((TILES * n,), jnp.int32),
        mesh=_sc_mesh(),
        scratch_types=[
            pltpu.VMEM((np_,), jnp.int32),
            pltpu.VMEM((et,), jnp.int32),
        ],
        compiler_params=_sc_params(),
    )
    def hist(pe_hbm, out_hbm, hist_v, ebuf):
        wid = lax.axis_index("s") * 2 + lax.axis_index("c")
        zeros = jnp.zeros((LANES,), jnp.int32)
        ones = jnp.ones((LANES,), jnp.int32)

        @pl.loop(0, np_, step=STEP)
        def _(i):
            for u in range(UNROLL):
                hist_v[pl.ds(i + u * LANES, LANES)] = zeros

        pltpu.sync_copy(pe_hbm.at[pl.ds(wid * et, et)], ebuf)

        @pl.loop(0, et, step=STEP)
        def _(i):
            for u in range(UNROLL):
                pe = ebuf[pl.ds(i + u * LANES, LANES)]
                d = (pe >> 16) & 0xFFFF
                plsc.addupdate_scatter(hist_v, [d], ones)

        pltpu.sync_copy(hist_v.at[pl.ds(0, n)], out_hbm.at[pl.ds(wid * n, n)])

    return hist


def _make_agg(f_dim: int, n: int, np_: int, ep: int, ce: int):
    nch = ep // ce
    npass = (f_dim + TILES - 1) // TILES

    @functools.partial(
        pl.kernel,
        out_type=jax.ShapeDtypeStruct((f_dim * n,), jnp.float32),
        mesh=_sc_mesh(),
        scratch_types=[
            pltpu.VMEM((np_,), jnp.float32),
            pltpu.VMEM((np_,), jnp.float32),
            pltpu.VMEM((ce,), jnp.int32),
            pltpu.VMEM((ce,), jnp.int32),
            pltpu.SemaphoreType.DMA,
            pltpu.SemaphoreType.DMA,
        ],
        compiler_params=_sc_params(),
    )
    def agg(y_hbm, pe_hbm, out_hbm, col_v, acc_v, eb0, eb1, sem0, sem1):
        wid = lax.axis_index("s") * 2 + lax.axis_index("c")
        zeros = jnp.zeros((LANES,), jnp.float32)

        def consume(ebuf):
            @pl.loop(0, ce, step=STEP)
            def _(i):
                for u in range(UNROLL):
                    pe = ebuf[pl.ds(i + u * LANES, LANES)]
                    s = pe & 0xFFFF
                    d = (pe >> 16) & 0xFFFF
                    v = plsc.load_gather(col_v, [s])
                    plsc.addupdate_scatter(acc_v, [d], v)

        for p in range(npass):
            feat = wid + TILES * p

            @pl.when(feat < f_dim)
            def _():
                pltpu.sync_copy(y_hbm.at[pl.ds(feat * n, n)], col_v.at[pl.ds(0, n)])

                @pl.loop(n, np_, step=LANES)
                def _(i):
                    col_v[pl.ds(i, LANES)] = zeros

                @pl.loop(0, np_, step=STEP)
                def _(i):
                    for u in range(UNROLL):
                        acc_v[pl.ds(i + u * LANES, LANES)] = zeros

                pltpu.async_copy(pe_hbm.at[pl.ds(0, ce)], eb0, sem0)

                @pl.loop(0, nch, step=2)
                def _(c):
                    pltpu.async_copy(pe_hbm.at[pl.ds((c + 1) * ce, ce)], eb1, sem1)
                    pltpu.make_async_copy(
                        pe_hbm.at[pl.ds(c * ce, ce)], eb0, sem0
                    ).wait()
                    consume(eb0)

                    @pl.when(c + 2 < nch)
                    def _():
                        pltpu.async_copy(
                            pe_hbm.at[pl.ds((c + 2) * ce, ce)], eb0, sem0
                        )

                    pltpu.make_async_copy(
                        pe_hbm.at[pl.ds((c + 1) * ce, ce)], eb1, sem1
                    ).wait()
                    consume(eb1)

                pltpu.sync_copy(acc_v.at[pl.ds(0, n)], out_hbm.at[pl.ds(feat * n, n)])

    return agg


_HIGHEST = jax.lax.Precision.HIGHEST


def _mm1(x, w1, bn):
    n, k_dim = x.shape
    h_dim = w1.shape[1]

    def body(x_ref, w_ref, o_ref):
        o_ref[...] = lax.dot_general(
            x_ref[...],
            w_ref[...],
            (((1,), (0,)), ((), ())),
            precision=_HIGHEST,
            preferred_element_type=jnp.float32,
        )

    return pl.pallas_call(
        body,
        grid=(n // bn,),
        in_specs=[
            pl.BlockSpec((bn, k_dim), lambda i: (i, 0)),
            pl.BlockSpec((k_dim, h_dim), lambda i: (0, 0)),
        ],
        out_specs=pl.BlockSpec((bn, h_dim), lambda i: (i, 0)),
        out_shape=jax.ShapeDtypeStruct((n, h_dim), jnp.float32),
    )(x, w1)


def _scale(hist_t, xw1, bn):
    n, h_dim = xw1.shape

    def body(hist_ref, xw_ref, y_ref, dinv_ref):
        deg = jnp.sum(hist_ref[...].astype(jnp.float32), axis=1, keepdims=True)
        dinv = lax.rsqrt(deg + 1.0)
        dinv_ref[...] = dinv
        y_ref[...] = xw_ref[...] * dinv

    return pl.pallas_call(
        body,
        grid=(n // bn,),
        in_specs=[
            pl.BlockSpec((bn, TILES), lambda i: (i, 0)),
            pl.BlockSpec((bn, h_dim), lambda i: (i, 0)),
        ],
        out_specs=[
            pl.BlockSpec((bn, h_dim), lambda i: (i, 0)),
            pl.BlockSpec((bn, 1), lambda i: (i, 0)),
        ],
        out_shape=[
            jax.ShapeDtypeStruct((n, h_dim), jnp.float32),
            jax.ShapeDtypeStruct((n, 1), jnp.float32),
        ],
    )(hist_t, xw1)


def _layer2(agg1, y1, dinv, b1_row, w2, bn):
    n, h_dim = y1.shape
    c_dim = w2.shape[1]

    def body(agg_ref, y1_ref, dinv_ref, b1_ref, w2_ref, y2_ref):
        dinv_blk = dinv_ref[...]
        h = jnp.maximum(
            dinv_blk * (agg_ref[...] + y1_ref[...]) + b1_ref[...], 0.0
        )
        y2 = lax.dot_general(
            h,
            w2_ref[...],
            (((1,), (0,)), ((), ())),
            precision=_HIGHEST,
            preferred_element_type=jnp.float32,
        )
        y2_ref[...] = y2 * dinv_blk

    return pl.pallas_call(
        body,
        grid=(n // bn,),
        in_specs=[
            pl.BlockSpec((bn, h_dim), lambda i: (i, 0)),
            pl.BlockSpec((bn, h_dim), lambda i: (i, 0)),
            pl.BlockSpec((bn, 1), lambda i: (i, 0)),
            pl.BlockSpec((1, h_dim), lambda i: (0, 0)),
            pl.BlockSpec((h_dim, c_dim), lambda i: (0, 0)),
        ],
        out_specs=pl.BlockSpec((bn, c_dim), lambda i: (i, 0)),
        out_shape=jax.ShapeDtypeStruct((n, c_dim), jnp.float32),
    )(agg1, y1, dinv, b1_row, w2)


def _final(agg2, y2, dinv, b2_row, bn):
    n, c_dim = y2.shape

    def body(agg_ref, y2_ref, dinv_ref, b2_ref, o_ref):
        o_ref[...] = (
            dinv_ref[...] * (agg_ref[...] + y2_ref[...]) + b2_ref[...]
        )

    return pl.pallas_call(
        body,
        grid=(n // bn,),
        in_specs=[
            pl.BlockSpec((bn, c_dim), lambda i: (i, 0)),
            pl.BlockSpec((bn, c_dim), lambda i: (i, 0)),
            pl.BlockSpec((bn, 1), lambda i: (i, 0)),
            pl.BlockSpec((1, c_dim), lambda i: (0, 0)),
        ],
        out_specs=pl.BlockSpec((bn, c_dim), lambda i: (i, 0)),
        out_shape=jax.ShapeDtypeStruct((n, c_dim), jnp.float32),
    )(agg2, y2, dinv, b2_row)


def kernel(x, edge_index, W1, b1, W2, b2):
    n, f_in = x.shape
    h_dim = W1.shape[1]
    c_dim = W2.shape[1]
    e = edge_index.shape[1]

    src = edge_index[0].astype(jnp.int32)
    dst = edge_index[1].astype(jnp.int32)
    pe = jnp.bitwise_or(src, jnp.left_shift(dst, 16))

    group = 2 * CE
    ep = ((e + group - 1) // group) * group
    if ep != e:
        sent = jnp.full((ep - e,), n, jnp.int32)
        pe = jnp.concatenate([pe, jnp.bitwise_or(sent, jnp.left_shift(sent, 16))])
    np_ = ((n + LANES + STEP - 1) // STEP) * STEP
    ce = CE
    assert 2 * np_ + 2 * ce <= 131000 and ep % TILES == 0 and (ep // TILES) % STEP == 0

    bn = 2000 if n % 2000 == 0 else max(
        b for b in (1000, 500, 250, 200, 100, 50, 40, 25, 8) if n % b == 0
    )

    hist = _make_hist(n, np_, ep)(pe).reshape(TILES, n)
    xw1 = _mm1(x, W1, bn)
    y1, dinv = _scale(hist.T, xw1, bn)
    agg1_t = _make_agg(h_dim, n, np_, ep, ce)(y1.T.reshape(-1), pe)
    agg1 = agg1_t.reshape(h_dim, n).T
    y2 = _layer2(agg1, y1, dinv, b1.reshape(1, h_dim), W2, bn)
    agg2_t = _make_agg(c_dim, n, np_, ep, ce)(y2.T.reshape(-1), pe)
    agg2 = agg2_t.reshape(c_dim, n).T
    return _final(agg2, y2, dinv, b2.reshape(1, c_dim), bn)


# speedup vs baseline: 17.4747x; 2.1366x over previous
import dataclasses
import functools

import jax
import jax.numpy as jnp
from jax import lax
from jax.experimental import pallas as pl
from jax.experimental.pallas import tpu as pltpu
from jax.experimental.pallas import tpu_sc as plsc

LANES = 16
TILES = 32


def _sc_mesh():
    return plsc.VectorSubcoreMesh(
        core_axis_name="c", subcore_axis_name="s", num_cores=2, num_subcores=16
    )


def _sc_params():
    cp = pltpu.CompilerParams()
    if "needs_layout_passes" in pltpu.CompilerParams.__dataclass_fields__:
        cp = dataclasses.replace(cp, needs_layout_passes=False)
    return cp


UNROLL = 8
STEP = UNROLL * LANES
CE = 8192


def _make_hist(n: int, np_: int, ep: int):
    et = ep // TILES

    @functools.partial(
        pl.kernel,
        out_type=jax.ShapeDtypeStruct((TILES * n,), jnp.int32),
        mesh=_sc_mesh(),
        scratch_types=[
            pltpu.VMEM((np_,), jnp.int32),
            pltpu.VMEM((et,), jnp.int32),
        ],
        compiler_params=_sc_params(),
    )
    def hist(pe_hbm, out_hbm, hist_v, ebuf):
        wid = lax.axis_index("s") * 2 + lax.axis_index("c")
        zeros = jnp.zeros((LANES,), jnp.int32)
        ones = jnp.ones((LANES,), jnp.int32)

        @pl.loop(0, np_, step=STEP)
        def _(i):
            for u in range(UNROLL):
                hist_v[pl.ds(i + u * LANES, LANES)] = zeros

        pltpu.sync_copy(pe_hbm.at[pl.ds(wid * et, et)], ebuf)

        @plsc.parallel_loop(0, et, LANES, unroll=UNROLL)
        def _(i):
            pe = ebuf[pl.ds(i, LANES)]
            d = (pe >> 16) & 0xFFFF
            plsc.addupdate_scatter(hist_v, [d], ones)

        pltpu.sync_copy(hist_v.at[pl.ds(0, n)], out_hbm.at[pl.ds(wid * n, n)])

    return hist


def _make_agg(f_dim: int, n: int, np_: int, ep: int, ce: int):
    nch = ep // ce
    npass = (f_dim + TILES - 1) // TILES

    @functools.partial(
        pl.kernel,
        out_type=jax.ShapeDtypeStruct((f_dim * n,), jnp.float32),
        mesh=_sc_mesh(),
        scratch_types=[
            pltpu.VMEM((np_,), jnp.float32),
            pltpu.VMEM((np_,), jnp.float32),
            pltpu.VMEM((ce,), jnp.int32),
            pltpu.VMEM((ce,), jnp.int32),
            pltpu.SemaphoreType.DMA,
            pltpu.SemaphoreType.DMA,
        ],
        compiler_params=_sc_params(),
    )
    def agg(y_hbm, pe_hbm, out_hbm, col_v, acc_v, eb0, eb1, sem0, sem1):
        wid = lax.axis_index("s") * 2 + lax.axis_index("c")
        zeros = jnp.zeros((LANES,), jnp.float32)

        def consume(ebuf):
            @plsc.parallel_loop(0, ce, LANES, unroll=UNROLL)
            def _(i):
                pe = ebuf[pl.ds(i, LANES)]
                s = pe & 0xFFFF
                d = (pe >> 16) & 0xFFFF
                v = plsc.load_gather(col_v, [s])
                plsc.addupdate_scatter(acc_v, [d], v)

        for p in range(npass):
            feat = wid + TILES * p

            @pl.when(feat < f_dim)
            def _():
                pltpu.sync_copy(y_hbm.at[pl.ds(feat * n, n)], col_v.at[pl.ds(0, n)])

                @pl.loop(n, np_, step=LANES)
                def _(i):
                    col_v[pl.ds(i, LANES)] = zeros

                @plsc.parallel_loop(0, np_, LANES, unroll=UNROLL)
                def _(i):
                    acc_v[pl.ds(i, LANES)] = zeros

                pltpu.async_copy(pe_hbm.at[pl.ds(0, ce)], eb0, sem0)

                @pl.loop(0, nch, step=2)
                def _(c):
                    pltpu.async_copy(pe_hbm.at[pl.ds((c + 1) * ce, ce)], eb1, sem1)
                    pltpu.make_async_copy(
                        pe_hbm.at[pl.ds(c * ce, ce)], eb0, sem0
                    ).wait()
                    consume(eb0)

                    @pl.when(c + 2 < nch)
                    def _():
                        pltpu.async_copy(
                            pe_hbm.at[pl.ds((c + 2) * ce, ce)], eb0, sem0
                        )

                    pltpu.make_async_copy(
                        pe_hbm.at[pl.ds((c + 1) * ce, ce)], eb1, sem1
                    ).wait()
                    consume(eb1)

                pltpu.sync_copy(acc_v.at[pl.ds(0, n)], out_hbm.at[pl.ds(feat * n, n)])

    return agg


_HIGHEST = jax.lax.Precision.HIGHEST


def _mm1(x, w1, bn):
    n, k_dim = x.shape
    h_dim = w1.shape[1]

    def body(x_ref, w_ref, o_ref):
        o_ref[...] = lax.dot_general(
            x_ref[...],
            w_ref[...],
            (((1,), (0,)), ((), ())),
            precision=_HIGHEST,
            preferred_element_type=jnp.float32,
        )

    return pl.pallas_call(
        body,
        grid=(n // bn,),
        in_specs=[
            pl.BlockSpec((bn, k_dim), lambda i: (i, 0)),
            pl.BlockSpec((k_dim, h_dim), lambda i: (0, 0)),
        ],
        out_specs=pl.BlockSpec((bn, h_dim), lambda i: (i, 0)),
        out_shape=jax.ShapeDtypeStruct((n, h_dim), jnp.float32),
    )(x, w1)


def _scale(hist_t, xw1, bn):
    n, h_dim = xw1.shape

    def body(hist_ref, xw_ref, y_ref, dinv_ref):
        deg = jnp.sum(hist_ref[...].astype(jnp.float32), axis=1, keepdims=True)
        dinv = lax.rsqrt(deg + 1.0)
        dinv_ref[...] = dinv
        y_ref[...] = xw_ref[...] * dinv

    return pl.pallas_call(
        body,
        grid=(n // bn,),
        in_specs=[
            pl.BlockSpec((bn, TILES), lambda i: (i, 0)),
            pl.BlockSpec((bn, h_dim), lambda i: (i, 0)),
        ],
        out_specs=[
            pl.BlockSpec((bn, h_dim), lambda i: (i, 0)),
            pl.BlockSpec((bn, 1), lambda i: (i, 0)),
        ],
        out_shape=[
            jax.ShapeDtypeStruct((n, h_dim), jnp.float32),
            jax.ShapeDtypeStruct((n, 1), jnp.float32),
        ],
    )(hist_t, xw1)


def _layer2(agg1, y1, dinv, b1_row, w2, bn):
    n, h_dim = y1.shape
    c_dim = w2.shape[1]

    def body(agg_ref, y1_ref, dinv_ref, b1_ref, w2_ref, y2_ref):
        dinv_blk = dinv_ref[...]
        h = jnp.maximum(
            dinv_blk * (agg_ref[...] + y1_ref[...]) + b1_ref[...], 0.0
        )
        y2 = lax.dot_general(
            h,
            w2_ref[...],
            (((1,), (0,)), ((), ())),
            precision=_HIGHEST,
            preferred_element_type=jnp.float32,
        )
        y2_ref[...] = y2 * dinv_blk

    return pl.pallas_call(
        body,
        grid=(n // bn,),
        in_specs=[
            pl.BlockSpec((bn, h_dim), lambda i: (i, 0)),
            pl.BlockSpec((bn, h_dim), lambda i: (i, 0)),
            pl.BlockSpec((bn, 1), lambda i: (i, 0)),
            pl.BlockSpec((1, h_dim), lambda i: (0, 0)),
            pl.BlockSpec((h_dim, c_dim), lambda i: (0, 0)),
        ],
        out_specs=pl.BlockSpec((bn, c_dim), lambda i: (i, 0)),
        out_shape=jax.ShapeDtypeStruct((n, c_dim), jnp.float32),
    )(agg1, y1, dinv, b1_row, w2)


def _final(agg2, y2, dinv, b2_row, bn):
    n, c_dim = y2.shape

    def body(agg_ref, y2_ref, dinv_ref, b2_ref, o_ref):
        o_ref[...] = (
            dinv_ref[...] * (agg_ref[...] + y2_ref[...]) + b2_ref[...]
        )

    return pl.pallas_call(
        body,
        grid=(n // bn,),
        in_specs=[
            pl.BlockSpec((bn, c_dim), lambda i: (i, 0)),
            pl.BlockSpec((bn, c_dim), lambda i: (i, 0)),
            pl.BlockSpec((bn, 1), lambda i: (i, 0)),
            pl.BlockSpec((1, c_dim), lambda i: (0, 0)),
        ],
        out_specs=pl.BlockSpec((bn, c_dim), lambda i: (i, 0)),
        out_shape=jax.ShapeDtypeStruct((n, c_dim), jnp.float32),
    )(agg2, y2, dinv, b2_row)


def kernel(x, edge_index, W1, b1, W2, b2):
    n, f_in = x.shape
    h_dim = W1.shape[1]
    c_dim = W2.shape[1]
    e = edge_index.shape[1]

    src = edge_index[0].astype(jnp.int32)
    dst = edge_index[1].astype(jnp.int32)
    pe = jnp.bitwise_or(src, jnp.left_shift(dst, 16))

    group = 2 * CE
    ep = ((e + group - 1) // group) * group
    if ep != e:
        sent = jnp.full((ep - e,), n, jnp.int32)
        pe = jnp.concatenate([pe, jnp.bitwise_or(sent, jnp.left_shift(sent, 16))])
    np_ = ((n + LANES + STEP - 1) // STEP) * STEP
    ce = CE
    assert 2 * np_ + 2 * ce <= 131000 and ep % TILES == 0 and (ep // TILES) % STEP == 0

    bn = 2000 if n % 2000 == 0 else max(
        b for b in (1000, 500, 250, 200, 100, 50, 40, 25, 8) if n % b == 0
    )

    hist = _make_hist(n, np_, ep)(pe).reshape(TILES, n)
    xw1 = _mm1(x, W1, bn)
    y1, dinv = _scale(hist.T, xw1, bn)
    agg1_t = _make_agg(h_dim, n, np_, ep, ce)(y1.T.reshape(-1), pe)
    agg1 = agg1_t.reshape(h_dim, n).T
    y2 = _layer2(agg1, y1, dinv, b1.reshape(1, h_dim), W2, bn)
    agg2_t = _make_agg(c_dim, n, np_, ep, ce)(y2.T.reshape(-1), pe)
    agg2 = agg2_t.reshape(c_dim, n).T
    return _final(agg2, y2, dinv, b2.reshape(1, c_dim), bn)


# speedup vs baseline: 19.7413x; 1.1297x over previous
import dataclasses
import functools

import jax
import jax.numpy as jnp
from jax import lax
from jax.experimental import pallas as pl
from jax.experimental.pallas import tpu as pltpu
from jax.experimental.pallas import tpu_sc as plsc

LANES = 16
TILES = 32


def _sc_mesh():
    return plsc.VectorSubcoreMesh(
        core_axis_name="c", subcore_axis_name="s", num_cores=2, num_subcores=16
    )


def _sc_params():
    cp = pltpu.CompilerParams()
    if "needs_layout_passes" in pltpu.CompilerParams.__dataclass_fields__:
        cp = dataclasses.replace(cp, needs_layout_passes=False)
    return cp


UNROLL = 8
STEP = UNROLL * LANES
CE = 8192


def _make_hist(n: int, np_: int, ep: int):
    et = ep // TILES

    @functools.partial(
        pl.kernel,
        out_type=jax.ShapeDtypeStruct((TILES * n,), jnp.int32),
        mesh=_sc_mesh(),
        scratch_types=[
            pltpu.VMEM((np_,), jnp.int32),
            pltpu.VMEM((et,), jnp.int32),
        ],
        compiler_params=_sc_params(),
    )
    def hist(pe_hbm, out_hbm, hist_v, ebuf):
        wid = lax.axis_index("s") * 2 + lax.axis_index("c")
        zeros = jnp.zeros((LANES,), jnp.int32)
        ones = jnp.ones((LANES,), jnp.int32)

        @pl.loop(0, np_, step=STEP)
        def _(i):
            for u in range(UNROLL):
                hist_v[pl.ds(i + u * LANES, LANES)] = zeros

        pltpu.sync_copy(pe_hbm.at[pl.ds(wid * et, et)], ebuf)

        @plsc.parallel_loop(0, et, LANES, unroll=UNROLL)
        def _(i):
            pe = ebuf[pl.ds(i, LANES)]
            d = (pe >> 16) & 0xFFFF
            plsc.addupdate_scatter(hist_v, [d], ones)

        pltpu.sync_copy(hist_v.at[pl.ds(0, n)], out_hbm.at[pl.ds(wid * n, n)])

    return hist


def _make_agg(f_dim: int, n: int, np_: int, ep: int, ce: int):
    nch = ep // ce
    npass = (f_dim + TILES - 1) // TILES

    @functools.partial(
        pl.kernel,
        out_type=jax.ShapeDtypeStruct((f_dim * n,), jnp.float32),
        mesh=_sc_mesh(),
        scratch_types=[
            pltpu.VMEM((np_,), jnp.float32),
            pltpu.VMEM((np_,), jnp.float32),
            pltpu.VMEM((ce,), jnp.int32),
            pltpu.VMEM((ce,), jnp.int32),
            pltpu.SemaphoreType.DMA,
            pltpu.SemaphoreType.DMA,
        ],
        compiler_params=_sc_params(),
    )
    def agg(y_hbm, pe_hbm, out_hbm, col_v, acc_v, eb0, eb1, sem0, sem1):
        wid = lax.axis_index("s") * 2 + lax.axis_index("c")
        zeros = jnp.zeros((LANES,), jnp.float32)

        def consume(ebuf):
            @plsc.parallel_loop(0, ce, LANES, unroll=UNROLL)
            def _(i):
                pe = ebuf[pl.ds(i, LANES)]
                s = pe & 0xFFFF
                d = (pe >> 16) & 0xFFFF
                v = plsc.load_gather(col_v, [s])
                plsc.addupdate_scatter(acc_v, [d], v)

        for p in range(npass):
            feat = wid + TILES * p

            @pl.when(feat < f_dim)
            def _():
                pltpu.sync_copy(y_hbm.at[pl.ds(feat * n, n)], col_v.at[pl.ds(0, n)])

                @pl.loop(n, np_, step=LANES)
                def _(i):
                    col_v[pl.ds(i, LANES)] = zeros

                @plsc.parallel_loop(0, np_, LANES, unroll=UNROLL)
                def _(i):
                    acc_v[pl.ds(i, LANES)] = zeros

                pltpu.async_copy(pe_hbm.at[pl.ds(0, ce)], eb0, sem0)

                @pl.loop(0, nch, step=2)
                def _(c):
                    pltpu.async_copy(pe_hbm.at[pl.ds((c + 1) * ce, ce)], eb1, sem1)
                    pltpu.make_async_copy(
                        pe_hbm.at[pl.ds(c * ce, ce)], eb0, sem0
                    ).wait()
                    consume(eb0)

                    @pl.when(c + 2 < nch)
                    def _():
                        pltpu.async_copy(
                            pe_hbm.at[pl.ds((c + 2) * ce, ce)], eb0, sem0
                        )

                    pltpu.make_async_copy(
                        pe_hbm.at[pl.ds((c + 1) * ce, ce)], eb1, sem1
                    ).wait()
                    consume(eb1)

                pltpu.sync_copy(acc_v.at[pl.ds(0, n)], out_hbm.at[pl.ds(feat * n, n)])

    return agg


_HIGHEST = jax.lax.Precision.HIGHEST


def _mm1(x, w1, bn):
    n, k_dim = x.shape
    h_dim = w1.shape[1]

    def body(x_ref, w_ref, o_ref):
        o_ref[...] = lax.dot_general(
            x_ref[...],
            w_ref[...],
            (((1,), (0,)), ((), ())),
            precision=jax.lax.Precision.DEFAULT,
            preferred_element_type=jnp.float32,
        )

    return pl.pallas_call(
        body,
        grid=(n // bn,),
        in_specs=[
            pl.BlockSpec((bn, k_dim), lambda i: (i, 0)),
            pl.BlockSpec((k_dim, h_dim), lambda i: (0, 0)),
        ],
        out_specs=pl.BlockSpec((bn, h_dim), lambda i: (i, 0)),
        out_shape=jax.ShapeDtypeStruct((n, h_dim), jnp.float32),
    )(x, w1)


def _scale(hist_t, xw1, bn):
    n, h_dim = xw1.shape

    def body(hist_ref, xw_ref, y_ref, dinv_ref):
        deg = jnp.sum(hist_ref[...].astype(jnp.float32), axis=1, keepdims=True)
        dinv = lax.rsqrt(deg + 1.0)
        dinv_ref[...] = dinv
        y_ref[...] = xw_ref[...] * dinv

    return pl.pallas_call(
        body,
        grid=(n // bn,),
        in_specs=[
            pl.BlockSpec((bn, TILES), lambda i: (i, 0)),
            pl.BlockSpec((bn, h_dim), lambda i: (i, 0)),
        ],
        out_specs=[
            pl.BlockSpec((bn, h_dim), lambda i: (i, 0)),
            pl.BlockSpec((bn, 1), lambda i: (i, 0)),
        ],
        out_shape=[
            jax.ShapeDtypeStruct((n, h_dim), jnp.float32),
            jax.ShapeDtypeStruct((n, 1), jnp.float32),
        ],
    )(hist_t, xw1)


def _layer2(agg1, y1, dinv, b1_row, w2, bn):
    n, h_dim = y1.shape
    c_dim = w2.shape[1]

    def body(agg_ref, y1_ref, dinv_ref, b1_ref, w2_ref, y2_ref):
        dinv_blk = dinv_ref[...]
        h = jnp.maximum(
            dinv_blk * (agg_ref[...] + y1_ref[...]) + b1_ref[...], 0.0
        )
        y2 = lax.dot_general(
            h,
            w2_ref[...],
            (((1,), (0,)), ((), ())),
            precision=_HIGHEST,
            preferred_element_type=jnp.float32,
        )
        y2_ref[...] = y2 * dinv_blk

    return pl.pallas_call(
        body,
        grid=(n // bn,),
        in_specs=[
            pl.BlockSpec((bn, h_dim), lambda i: (i, 0)),
            pl.BlockSpec((bn, h_dim), lambda i: (i, 0)),
            pl.BlockSpec((bn, 1), lambda i: (i, 0)),
            pl.BlockSpec((1, h_dim), lambda i: (0, 0)),
            pl.BlockSpec((h_dim, c_dim), lambda i: (0, 0)),
        ],
        out_specs=pl.BlockSpec((bn, c_dim), lambda i: (i, 0)),
        out_shape=jax.ShapeDtypeStruct((n, c_dim), jnp.float32),
    )(agg1, y1, dinv, b1_row, w2)


def _final(agg2, y2, dinv, b2_row, bn):
    n, c_dim = y2.shape

    def body(agg_ref, y2_ref, dinv_ref, b2_ref, o_ref):
        o_ref[...] = (
            dinv_ref[...] * (agg_ref[...] + y2_ref[...]) + b2_ref[...]
        )

    return pl.pallas_call(
        body,
        grid=(n // bn,),
        in_specs=[
            pl.BlockSpec((bn, c_dim), lambda i: (i, 0)),
            pl.BlockSpec((bn, c_dim), lambda i: (i, 0)),
            pl.BlockSpec((bn, 1), lambda i: (i, 0)),
            pl.BlockSpec((1, c_dim), lambda i: (0, 0)),
        ],
        out_specs=pl.BlockSpec((bn, c_dim), lambda i: (i, 0)),
        out_shape=jax.ShapeDtypeStruct((n, c_dim), jnp.float32),
    )(agg2, y2, dinv, b2_row)


def kernel(x, edge_index, W1, b1, W2, b2):
    n, f_in = x.shape
    h_dim = W1.shape[1]
    c_dim = W2.shape[1]
    e = edge_index.shape[1]

    src = edge_index[0].astype(jnp.int32)
    dst = edge_index[1].astype(jnp.int32)
    pe = jnp.bitwise_or(src, jnp.left_shift(dst, 16))

    group = 2 * CE
    ep = ((e + group - 1) // group) * group
    if ep != e:
        sent = jnp.full((ep - e,), n, jnp.int32)
        pe = jnp.concatenate([pe, jnp.bitwise_or(sent, jnp.left_shift(sent, 16))])
    np_ = ((n + LANES + STEP - 1) // STEP) * STEP
    ce = CE
    assert 2 * np_ + 2 * ce <= 131000 and ep % TILES == 0 and (ep // TILES) % STEP == 0

    bn = 2000 if n % 2000 == 0 else max(
        b for b in (1000, 500, 250, 200, 100, 50, 40, 25, 8) if n % b == 0
    )

    hist = _make_hist(n, np_, ep)(pe).reshape(TILES, n)
    xw1 = _mm1(x, W1, bn)
    y1, dinv = _scale(hist.T, xw1, bn)
    agg1_t = _make_agg(h_dim, n, np_, ep, ce)(y1.T.reshape(-1), pe)
    agg1 = agg1_t.reshape(h_dim, n).T
    y2 = _layer2(agg1, y1, dinv, b1.reshape(1, h_dim), W2, bn)
    agg2_t = _make_agg(c_dim, n, np_, ep, ce)(y2.T.reshape(-1), pe)
    agg2 = agg2_t.reshape(c_dim, n).T
    return _final(agg2, y2, dinv, b2.reshape(1, c_dim), bn)


# speedup vs baseline: 20.0756x; 1.0169x over previous
import dataclasses
import functools

import jax
import jax.numpy as jnp
from jax import lax
from jax.experimental import pallas as pl
from jax.experimental.pallas import tpu as pltpu
from jax.experimental.pallas import tpu_sc as plsc

LANES = 16
TILES = 32


def _sc_mesh():
    return plsc.VectorSubcoreMesh(
        core_axis_name="c", subcore_axis_name="s", num_cores=2, num_subcores=16
    )


def _sc_params():
    cp = pltpu.CompilerParams()
    if "needs_layout_passes" in pltpu.CompilerParams.__dataclass_fields__:
        cp = dataclasses.replace(cp, needs_layout_passes=False)
    return cp


UNROLL = 8
STEP = UNROLL * LANES
CE = 8192


def _make_hist(n: int, np_: int, ep: int):
    et = ep // TILES

    @functools.partial(
        pl.kernel,
        out_type=jax.ShapeDtypeStruct((TILES * n,), jnp.int32),
        mesh=_sc_mesh(),
        scratch_types=[
            pltpu.VMEM((np_,), jnp.int32),
            pltpu.VMEM((et,), jnp.int32),
        ],
        compiler_params=_sc_params(),
    )
    def hist(pe_hbm, out_hbm, hist_v, ebuf):
        wid = lax.axis_index("s") * 2 + lax.axis_index("c")
        zeros = jnp.zeros((LANES,), jnp.int32)
        ones = jnp.ones((LANES,), jnp.int32)

        @pl.loop(0, np_, step=STEP)
        def _(i):
            for u in range(UNROLL):
                hist_v[pl.ds(i + u * LANES, LANES)] = zeros

        pltpu.sync_copy(pe_hbm.at[pl.ds(wid * et, et)], ebuf)

        @plsc.parallel_loop(0, et, LANES, unroll=UNROLL)
        def _(i):
            pe = ebuf[pl.ds(i, LANES)]
            d = (pe >> 16) & 0xFFFF
            plsc.addupdate_scatter(hist_v, [d], ones)

        pltpu.sync_copy(hist_v.at[pl.ds(0, n)], out_hbm.at[pl.ds(wid * n, n)])

    return hist


def _make_agg(out_rows: int, n: int, np_: int, ep: int, ce: int, passes):
    @functools.partial(
        pl.kernel,
        out_type=jax.ShapeDtypeStruct((out_rows * n,), jnp.float32),
        mesh=_sc_mesh(),
        scratch_types=[
            pltpu.VMEM((np_,), jnp.float32),
            pltpu.VMEM((np_,), jnp.float32),
            pltpu.VMEM((ce,), jnp.int32),
            pltpu.VMEM((ce,), jnp.int32),
            pltpu.SemaphoreType.DMA,
            pltpu.SemaphoreType.DMA,
        ],
        compiler_params=_sc_params(),
    )
    def agg(y_hbm, pe_hbm, out_hbm, col_v, acc_v, eb0, eb1, sem0, sem1):
        wid = lax.axis_index("s") * 2 + lax.axis_index("c")
        zeros = jnp.zeros((LANES,), jnp.float32)

        def consume(ebuf):
            @plsc.parallel_loop(0, ce, LANES, unroll=UNROLL)
            def _(i):
                pe = ebuf[pl.ds(i, LANES)]
                s = pe & 0xFFFF
                d = (pe >> 16) & 0xFFFF
                v = plsc.load_gather(col_v, [s])
                plsc.addupdate_scatter(acc_v, [d], v)

        for job in passes:
            feat, startc, nch, outrow = job(wid)
            assert nch % 2 == 0 and nch * ce <= ep

            pltpu.sync_copy(y_hbm.at[pl.ds(feat * n, n)], col_v.at[pl.ds(0, n)])

            @pl.loop(n, np_, step=LANES)
            def _(i):
                col_v[pl.ds(i, LANES)] = zeros

            @plsc.parallel_loop(0, np_, LANES, unroll=UNROLL)
            def _(i):
                acc_v[pl.ds(i, LANES)] = zeros

            pltpu.async_copy(pe_hbm.at[pl.ds(startc * ce, ce)], eb0, sem0)

            @pl.loop(0, nch, step=2)
            def _(c):
                c0 = (startc + c) * ce
                pltpu.async_copy(pe_hbm.at[pl.ds(c0 + ce, ce)], eb1, sem1)
                pltpu.make_async_copy(
                    pe_hbm.at[pl.ds(c0, ce)], eb0, sem0
                ).wait()
                consume(eb0)

                @pl.when(c + 2 < nch)
                def _():
                    pltpu.async_copy(
                        pe_hbm.at[pl.ds(c0 + 2 * ce, ce)], eb0, sem0
                    )

                pltpu.make_async_copy(
                    pe_hbm.at[pl.ds(c0 + ce, ce)], eb1, sem1
                ).wait()
                consume(eb1)

            pltpu.sync_copy(acc_v.at[pl.ds(0, n)], out_hbm.at[pl.ds(outrow * n, n)])

    return agg


_HIGHEST = jax.lax.Precision.HIGHEST


def _mm1(x, w1, bn):
    n, k_dim = x.shape
    h_dim = w1.shape[1]

    def body(x_ref, w_ref, o_ref):
        o_ref[...] = lax.dot_general(
            x_ref[...],
            w_ref[...],
            (((1,), (0,)), ((), ())),
            precision=jax.lax.Precision.DEFAULT,
            preferred_element_type=jnp.float32,
        )

    return pl.pallas_call(
        body,
        grid=(n // bn,),
        in_specs=[
            pl.BlockSpec((bn, k_dim), lambda i: (i, 0)),
            pl.BlockSpec((k_dim, h_dim), lambda i: (0, 0)),
        ],
        out_specs=pl.BlockSpec((bn, h_dim), lambda i: (i, 0)),
        out_shape=jax.ShapeDtypeStruct((n, h_dim), jnp.float32),
    )(x, w1)


def _scale(hist_t, xw1, bn):
    n, h_dim = xw1.shape

    def body(hist_ref, xw_ref, y_ref, dinv_ref):
        deg = jnp.sum(hist_ref[...].astype(jnp.float32), axis=1, keepdims=True)
        dinv = lax.rsqrt(deg + 1.0)
        dinv_ref[...] = dinv
        y_ref[...] = xw_ref[...] * dinv

    return pl.pallas_call(
        body,
        grid=(n // bn,),
        in_specs=[
            pl.BlockSpec((bn, TILES), lambda i: (i, 0)),
            pl.BlockSpec((bn, h_dim), lambda i: (i, 0)),
        ],
        out_specs=[
            pl.BlockSpec((bn, h_dim), lambda i: (i, 0)),
            pl.BlockSpec((bn, 1), lambda i: (i, 0)),
        ],
        out_shape=[
            jax.ShapeDtypeStruct((n, h_dim), jnp.float32),
            jax.ShapeDtypeStruct((n, 1), jnp.float32),
        ],
    )(hist_t, xw1)


def _layer2(agg1, y1, dinv, b1_row, w2, bn):
    n, h_dim = y1.shape
    c_dim = w2.shape[1]

    def body(agg_ref, y1_ref, dinv_ref, b1_ref, w2_ref, y2_ref):
        dinv_blk = dinv_ref[...]
        h = jnp.maximum(
            dinv_blk * (agg_ref[...] + y1_ref[...]) + b1_ref[...], 0.0
        )
        y2 = lax.dot_general(
            h,
            w2_ref[...],
            (((1,), (0,)), ((), ())),
            precision=_HIGHEST,
            preferred_element_type=jnp.float32,
        )
        y2_ref[...] = y2 * dinv_blk

    return pl.pallas_call(
        body,
        grid=(n // bn,),
        in_specs=[
            pl.BlockSpec((bn, h_dim), lambda i: (i, 0)),
            pl.BlockSpec((bn, h_dim), lambda i: (i, 0)),
            pl.BlockSpec((bn, 1), lambda i: (i, 0)),
            pl.BlockSpec((1, h_dim), lambda i: (0, 0)),
            pl.BlockSpec((h_dim, c_dim), lambda i: (0, 0)),
        ],
        out_specs=pl.BlockSpec((bn, c_dim), lambda i: (i, 0)),
        out_shape=jax.ShapeDtypeStruct((n, c_dim), jnp.float32),
    )(agg1, y1, dinv, b1_row, w2)


def _final(agg2, y2, dinv, b2_row, bn):
    n, c_dim = y2.shape

    def body(agg_ref, y2_ref, dinv_ref, b2_ref, o_ref):
        a = agg_ref[...]
        tail = (
            a[:, 32:40] + a[:, 40:48] + a[:, 48:56] + a[:, 56:64]
        )
        agg_full = jnp.concatenate([a[:, :32], tail], axis=1)
        o_ref[...] = (
            dinv_ref[...] * (agg_full + y2_ref[...]) + b2_ref[...]
        )

    return pl.pallas_call(
        body,
        grid=(n // bn,),
        in_specs=[
            pl.BlockSpec((bn, 64), lambda i: (i, 0)),
            pl.BlockSpec((bn, c_dim), lambda i: (i, 0)),
            pl.BlockSpec((bn, 1), lambda i: (i, 0)),
            pl.BlockSpec((1, c_dim), lambda i: (0, 0)),
        ],
        out_specs=pl.BlockSpec((bn, c_dim), lambda i: (i, 0)),
        out_shape=jax.ShapeDtypeStruct((n, c_dim), jnp.float32),
    )(agg2, y2, dinv, b2_row)


def kernel(x, edge_index, W1, b1, W2, b2):
    n, f_in = x.shape
    h_dim = W1.shape[1]
    c_dim = W2.shape[1]
    e = edge_index.shape[1]

    src = edge_index[0].astype(jnp.int32)
    dst = edge_index[1].astype(jnp.int32)
    pe = jnp.bitwise_or(src, jnp.left_shift(dst, 16))

    group = 2 * CE
    ep = ((e + group - 1) // group) * group
    if ep != e:
        sent = jnp.full((ep - e,), n, jnp.int32)
        pe = jnp.concatenate([pe, jnp.bitwise_or(sent, jnp.left_shift(sent, 16))])
    np_ = ((n + LANES + STEP - 1) // STEP) * STEP
    ce = CE
    assert 2 * np_ + 2 * ce <= 131000 and ep % TILES == 0 and (ep // TILES) % STEP == 0

    bn = 2000 if n % 2000 == 0 else max(
        b for b in (1000, 500, 250, 200, 100, 50, 40, 25, 8) if n % b == 0
    )

    nch_full = ep // ce
    ce2 = 2048
    nch_q = ep // (4 * ce2)
    assert ep % (8 * ce2) == 0

    l1_passes = [
        (lambda wid, p=p: (wid + TILES * p, 0, nch_full, wid + TILES * p))
        for p in range(h_dim // TILES)
    ]
    l2_passes = [
        lambda wid: (wid, 0, ep // ce2, wid),
        lambda wid: (
            32 + (wid >> 2),
            (wid & 3) * nch_q,
            nch_q,
            32 + 8 * (wid & 3) + (wid >> 2),
        ),
    ]

    hist = _make_hist(n, np_, ep)(pe).reshape(TILES, n)
    xw1 = _mm1(x, W1, bn)
    y1, dinv = _scale(hist.T, xw1, bn)
    agg1_t = _make_agg(h_dim, n, np_, ep, ce, l1_passes)(y1.T.reshape(-1), pe)
    agg1 = agg1_t.reshape(h_dim, n).T
    y2 = _layer2(agg1, y1, dinv, b1.reshape(1, h_dim), W2, bn)
    agg2_t = _make_agg(64, n, np_, ep, ce2, l2_passes)(y2.T.reshape(-1), pe)
    agg2 = agg2_t.reshape(64, n).T
    return _final(agg2, y2, dinv, b2.reshape(1, c_dim), bn)


# speedup vs baseline: 21.2235x; 1.0572x over previous
import dataclasses
import functools

import jax
import jax.numpy as jnp
from jax import lax
from jax.experimental import pallas as pl
from jax.experimental.pallas import tpu as pltpu
from jax.experimental.pallas import tpu_sc as plsc

LANES = 16
TILES = 32


def _sc_mesh():
    return plsc.VectorSubcoreMesh(
        core_axis_name="c", subcore_axis_name="s", num_cores=2, num_subcores=16
    )


def _sc_params():
    cp = pltpu.CompilerParams()
    if "needs_layout_passes" in pltpu.CompilerParams.__dataclass_fields__:
        cp = dataclasses.replace(cp, needs_layout_passes=False)
    return cp


UNROLL = 8
STEP = UNROLL * LANES
CE = 8192


def _make_hist(n: int, np_: int, ep: int):
    et = ep // TILES

    @functools.partial(
        pl.kernel,
        out_type=jax.ShapeDtypeStruct((TILES * n,), jnp.int32),
        mesh=_sc_mesh(),
        scratch_types=[
            pltpu.VMEM((np_,), jnp.int32),
            pltpu.VMEM((et,), jnp.int32),
        ],
        compiler_params=_sc_params(),
    )
    def hist(pe_hbm, out_hbm, hist_v, ebuf):
        wid = lax.axis_index("s") * 2 + lax.axis_index("c")
        zeros = jnp.zeros((LANES,), jnp.int32)
        ones = jnp.ones((LANES,), jnp.int32)

        @pl.loop(0, np_, step=STEP)
        def _(i):
            for u in range(UNROLL):
                hist_v[pl.ds(i + u * LANES, LANES)] = zeros

        pltpu.sync_copy(pe_hbm.at[pl.ds(wid * et, et)], ebuf)

        @plsc.parallel_loop(0, et, LANES, unroll=UNROLL)
        def _(i):
            pe = ebuf[pl.ds(i, LANES)]
            d = (pe >> 16) & 0xFFFF
            plsc.addupdate_scatter(hist_v, [d], ones)

        pltpu.sync_copy(hist_v.at[pl.ds(0, n)], out_hbm.at[pl.ds(wid * n, n)])

    return hist


def _make_agg(out_rows: int, n: int, np_: int, ep: int, ce: int, passes):
    @functools.partial(
        pl.kernel,
        out_type=jax.ShapeDtypeStruct((out_rows * n,), jnp.float32),
        mesh=_sc_mesh(),
        scratch_types=[
            pltpu.VMEM((np_,), jnp.float32),
            pltpu.VMEM((np_,), jnp.float32),
            pltpu.VMEM((ce,), jnp.int32),
            pltpu.VMEM((ce,), jnp.int32),
            pltpu.SemaphoreType.DMA,
            pltpu.SemaphoreType.DMA,
        ],
        compiler_params=_sc_params(),
    )
    def agg(y_hbm, pe_hbm, out_hbm, col_v, acc_v, eb0, eb1, sem0, sem1):
        wid = lax.axis_index("s") * 2 + lax.axis_index("c")
        zeros = jnp.zeros((LANES,), jnp.float32)

        for job, ce_p in passes:
            assert ce_p <= ce

            def consume(ebuf, ce_p=ce_p):
                @plsc.parallel_loop(0, ce_p, LANES, unroll=UNROLL)
                def _(i):
                    pe = ebuf[pl.ds(i, LANES)]
                    s = pe & 0xFFFF
                    d = (pe >> 16) & 0xFFFF
                    v = plsc.load_gather(col_v, [s])
                    plsc.addupdate_scatter(acc_v, [d], v)

            feat, startc, nch, outrow = job(wid)
            assert nch % 2 == 0 and nch * ce_p <= ep

            pltpu.sync_copy(y_hbm.at[pl.ds(feat * n, n)], col_v.at[pl.ds(0, n)])

            @pl.loop(n, np_, step=LANES)
            def _(i):
                col_v[pl.ds(i, LANES)] = zeros

            @plsc.parallel_loop(0, np_, LANES, unroll=UNROLL)
            def _(i):
                acc_v[pl.ds(i, LANES)] = zeros

            eb0s, eb1s = eb0.at[pl.ds(0, ce_p)], eb1.at[pl.ds(0, ce_p)]
            pltpu.async_copy(pe_hbm.at[pl.ds(startc * ce_p, ce_p)], eb0s, sem0)

            @pl.loop(0, nch, step=2)
            def _(c, ce_p=ce_p, eb0s=eb0s, eb1s=eb1s):
                c0 = (startc + c) * ce_p
                pltpu.async_copy(pe_hbm.at[pl.ds(c0 + ce_p, ce_p)], eb1s, sem1)
                pltpu.make_async_copy(
                    pe_hbm.at[pl.ds(c0, ce_p)], eb0s, sem0
                ).wait()
                consume(eb0s)

                @pl.when(c + 2 < nch)
                def _():
                    pltpu.async_copy(
                        pe_hbm.at[pl.ds(c0 + 2 * ce_p, ce_p)], eb0s, sem0
                    )

                pltpu.make_async_copy(
                    pe_hbm.at[pl.ds(c0 + ce_p, ce_p)], eb1s, sem1
                ).wait()
                consume(eb1s)

            pltpu.sync_copy(acc_v.at[pl.ds(0, n)], out_hbm.at[pl.ds(outrow * n, n)])

    return agg


_HIGHEST = jax.lax.Precision.HIGHEST


def _mm1(x, w1, bn):
    n, k_dim = x.shape
    h_dim = w1.shape[1]

    def body(x_ref, w_ref, o_ref):
        o_ref[...] = lax.dot_general(
            x_ref[...],
            w_ref[...],
            (((1,), (0,)), ((), ())),
            precision=jax.lax.Precision.DEFAULT,
            preferred_element_type=jnp.float32,
        )

    return pl.pallas_call(
        body,
        grid=(n // bn,),
        in_specs=[
            pl.BlockSpec((bn, k_dim), lambda i: (i, 0)),
            pl.BlockSpec((k_dim, h_dim), lambda i: (0, 0)),
        ],
        out_specs=pl.BlockSpec((bn, h_dim), lambda i: (i, 0)),
        out_shape=jax.ShapeDtypeStruct((n, h_dim), jnp.float32),
    )(x, w1)


def _scale(hist_t, xw1, bn):
    n, h_dim = xw1.shape

    def body(hist_ref, xw_ref, y_ref, dinv_ref):
        deg = jnp.sum(hist_ref[...].astype(jnp.float32), axis=1, keepdims=True)
        dinv = lax.rsqrt(deg + 1.0)
        dinv_ref[...] = dinv
        y_ref[...] = xw_ref[...] * dinv

    return pl.pallas_call(
        body,
        grid=(n // bn,),
        in_specs=[
            pl.BlockSpec((bn, TILES), lambda i: (i, 0)),
            pl.BlockSpec((bn, h_dim), lambda i: (i, 0)),
        ],
        out_specs=[
            pl.BlockSpec((bn, h_dim), lambda i: (i, 0)),
            pl.BlockSpec((bn, 1), lambda i: (i, 0)),
        ],
        out_shape=[
            jax.ShapeDtypeStruct((n, h_dim), jnp.float32),
            jax.ShapeDtypeStruct((n, 1), jnp.float32),
        ],
    )(hist_t, xw1)


def _layer2(agg1, y1, dinv, b1_row, w2, bn):
    n, h_dim = y1.shape
    c_dim = w2.shape[1]

    def body(agg_ref, y1_ref, dinv_ref, b1_ref, w2_ref, y2_ref):
        dinv_blk = dinv_ref[...]
        h = jnp.maximum(
            dinv_blk * (agg_ref[...] + y1_ref[...]) + b1_ref[...], 0.0
        )
        y2 = lax.dot_general(
            h,
            w2_ref[...],
            (((1,), (0,)), ((), ())),
            precision=_HIGHEST,
            preferred_element_type=jnp.float32,
        )
        y2_ref[...] = y2 * dinv_blk

    return pl.pallas_call(
        body,
        grid=(n // bn,),
        in_specs=[
            pl.BlockSpec((bn, h_dim), lambda i: (i, 0)),
            pl.BlockSpec((bn, h_dim), lambda i: (i, 0)),
            pl.BlockSpec((bn, 1), lambda i: (i, 0)),
            pl.BlockSpec((1, h_dim), lambda i: (0, 0)),
            pl.BlockSpec((h_dim, c_dim), lambda i: (0, 0)),
        ],
        out_specs=pl.BlockSpec((bn, c_dim), lambda i: (i, 0)),
        out_shape=jax.ShapeDtypeStruct((n, c_dim), jnp.float32),
    )(agg1, y1, dinv, b1_row, w2)


def _final(agg2, y2, dinv, b2_row, bn):
    n, c_dim = y2.shape

    def body(agg_ref, y2_ref, dinv_ref, b2_ref, o_ref):
        a = agg_ref[...]
        tail = (
            a[:, 32:40] + a[:, 40:48] + a[:, 48:56] + a[:, 56:64]
        )
        agg_full = jnp.concatenate([a[:, :32], tail], axis=1)
        o_ref[...] = (
            dinv_ref[...] * (agg_full + y2_ref[...]) + b2_ref[...]
        )

    return pl.pallas_call(
        body,
        grid=(n // bn,),
        in_specs=[
            pl.BlockSpec((bn, 64), lambda i: (i, 0)),
            pl.BlockSpec((bn, c_dim), lambda i: (i, 0)),
            pl.BlockSpec((bn, 1), lambda i: (i, 0)),
            pl.BlockSpec((1, c_dim), lambda i: (0, 0)),
        ],
        out_specs=pl.BlockSpec((bn, c_dim), lambda i: (i, 0)),
        out_shape=jax.ShapeDtypeStruct((n, c_dim), jnp.float32),
    )(agg2, y2, dinv, b2_row)


def kernel(x, edge_index, W1, b1, W2, b2):
    n, f_in = x.shape
    h_dim = W1.shape[1]
    c_dim = W2.shape[1]
    e = edge_index.shape[1]

    src = edge_index[0].astype(jnp.int32)
    dst = edge_index[1].astype(jnp.int32)
    pe = jnp.bitwise_or(src, jnp.left_shift(dst, 16))

    group = 2 * CE
    ep = ((e + group - 1) // group) * group
    if ep != e:
        sent = jnp.full((ep - e,), n, jnp.int32)
        pe = jnp.concatenate([pe, jnp.bitwise_or(sent, jnp.left_shift(sent, 16))])
    np_ = ((n + LANES + STEP - 1) // STEP) * STEP
    ce = CE
    assert 2 * np_ + 2 * ce <= 131000 and ep % TILES == 0 and (ep // TILES) % STEP == 0

    bn = 2000 if n % 2000 == 0 else max(
        b for b in (1000, 500, 250, 200, 100, 50, 40, 25, 8) if n % b == 0
    )

    nch_full = ep // ce
    ce2 = 2048
    nch_q = ep // (4 * ce2)
    assert ep % (8 * ce2) == 0

    l1_passes = [
        ((lambda wid, p=p: (wid + TILES * p, 0, nch_full, wid + TILES * p)), ce)
        for p in range(h_dim // TILES)
    ]
    l2_passes = [
        ((lambda wid: (wid, 0, nch_full, wid)), ce),
        (
            (lambda wid: (
                32 + (wid >> 2),
                (wid & 3) * nch_q,
                nch_q,
                32 + 8 * (wid & 3) + (wid >> 2),
            )),
            ce2,
        ),
    ]

    hist = _make_hist(n, np_, ep)(pe).reshape(TILES, n)
    xw1 = _mm1(x, W1, bn)
    y1, dinv = _scale(hist.T, xw1, bn)
    agg1_t = _make_agg(h_dim, n, np_, ep, ce, l1_passes)(y1.T.reshape(-1), pe)
    agg1 = agg1_t.reshape(h_dim, n).T
    y2 = _layer2(agg1, y1, dinv, b1.reshape(1, h_dim), W2, bn)
    agg2_t = _make_agg(64, n, np_, ep, ce, l2_passes)(y2.T.reshape(-1), pe)
    agg2 = agg2_t.reshape(64, n).T
    return _final(agg2, y2, dinv, b2.reshape(1, c_dim), bn)


# speedup vs baseline: 22.5357x; 1.0618x over previous
import dataclasses
import functools

import jax
import jax.numpy as jnp
from jax import lax
from jax.experimental import pallas as pl
from jax.experimental.pallas import tpu as pltpu
from jax.experimental.pallas import tpu_sc as plsc

LANES = 16
TILES = 32
UNROLL = 8
STEP = UNROLL * LANES
CE = 8192


def _sc_mesh():
    return plsc.VectorSubcoreMesh(
        core_axis_name="c", subcore_axis_name="s", num_cores=2, num_subcores=16
    )


def _sc_params():
    cp = pltpu.CompilerParams()
    if "needs_layout_passes" in pltpu.CompilerParams.__dataclass_fields__:
        cp = dataclasses.replace(cp, needs_layout_passes=False)
    return cp


def _seg_gather(hbm, col_v, feat, f_dim, nb, bn, bnp, sem):
    for b in range(nb):
        pltpu.async_copy(
            hbm.at[pl.ds((b * f_dim + feat) * bnp, bn)],
            col_v.at[pl.ds(b * bn, bn)],
            sem,
        )
    for b in range(nb):
        pltpu.make_async_copy(
            hbm.at[pl.ds((b * f_dim + feat) * bnp, bn)],
            col_v.at[pl.ds(b * bn, bn)],
            sem,
        ).wait()


def _seg_scatter(col_v, hbm, feat, f_dim, nb, bn, bnp, sem):
    for b in range(nb):
        pltpu.async_copy(
            col_v.at[pl.ds(b * bn, bn)],
            hbm.at[pl.ds((b * f_dim + feat) * bnp, bn)],
            sem,
        )
    for b in range(nb):
        pltpu.make_async_copy(
            col_v.at[pl.ds(b * bn, bn)],
            hbm.at[pl.ds((b * f_dim + feat) * bnp, bn)],
            sem,
        ).wait()


def _make_hist(n: int, np_: int, ep: int, nb: int, bn: int, bnp: int):
    et = ep // TILES

    @functools.partial(
        pl.kernel,
        out_type=jax.ShapeDtypeStruct((nb * TILES * bnp,), jnp.int32),
        mesh=_sc_mesh(),
        scratch_types=[
            pltpu.VMEM((np_,), jnp.int32),
            pltpu.VMEM((et,), jnp.int32),
            pltpu.SemaphoreType.DMA,
        ],
        compiler_params=_sc_params(),
    )
    def hist(pe_hbm, out_hbm, hist_v, ebuf, sem):
        wid = lax.axis_index("s") * 2 + lax.axis_index("c")
        zeros = jnp.zeros((LANES,), jnp.int32)
        ones = jnp.ones((LANES,), jnp.int32)

        @pl.loop(0, np_, step=STEP)
        def _(i):
            for u in range(UNROLL):
                hist_v[pl.ds(i + u * LANES, LANES)] = zeros

        pltpu.sync_copy(pe_hbm.at[pl.ds(wid * et, et)], ebuf)

        @plsc.parallel_loop(0, et, LANES, unroll=UNROLL)
        def _(i):
            pe = ebuf[pl.ds(i, LANES)]
            d = (pe >> 16) & 0xFFFF
            plsc.addupdate_scatter(hist_v, [d], ones)

        _seg_scatter(hist_v, out_hbm, wid, TILES, nb, bn, bnp, sem)

    return hist


def _make_agg(
    out_rows: int,
    in_feats: int,
    n: int,
    np_: int,
    ep: int,
    ce: int,
    nb: int,
    bn: int,
    bnp: int,
    passes,
):
    @functools.partial(
        pl.kernel,
        out_type=jax.ShapeDtypeStruct((nb * out_rows * bnp,), jnp.float32),
        mesh=_sc_mesh(),
        scratch_types=[
            pltpu.VMEM((np_,), jnp.float32),
            pltpu.VMEM((np_,), jnp.float32),
            pltpu.VMEM((ce,), jnp.int32),
            pltpu.VMEM((ce,), jnp.int32),
            pltpu.SemaphoreType.DMA,
            pltpu.SemaphoreType.DMA,
        ],
        compiler_params=_sc_params(),
    )
    def agg(y_hbm, pe_hbm, out_hbm, col_v, acc_v, eb0, eb1, sem0, sem1):
        wid = lax.axis_index("s") * 2 + lax.axis_index("c")
        zeros = jnp.zeros((LANES,), jnp.float32)

        for job, ce_p in passes:
            assert ce_p <= ce

            def consume(ebuf, ce_p=ce_p):
                @plsc.parallel_loop(0, ce_p, LANES, unroll=UNROLL)
                def _(i):
                    pe = ebuf[pl.ds(i, LANES)]
                    s = pe & 0xFFFF
                    d = (pe >> 16) & 0xFFFF
                    v = plsc.load_gather(col_v, [s])
                    plsc.addupdate_scatter(acc_v, [d], v)

            feat, startc, nch, outrow = job(wid)
            assert nch % 2 == 0 and nch * ce_p <= ep

            _seg_gather(y_hbm, col_v, feat, in_feats, nb, bn, bnp, sem0)

            @pl.loop(n, np_, step=LANES)
            def _(i):
                col_v[pl.ds(i, LANES)] = zeros

            @plsc.parallel_loop(0, np_, LANES, unroll=UNROLL)
            def _(i):
                acc_v[pl.ds(i, LANES)] = zeros

            eb0s, eb1s = eb0.at[pl.ds(0, ce_p)], eb1.at[pl.ds(0, ce_p)]
            pltpu.async_copy(pe_hbm.at[pl.ds(startc * ce_p, ce_p)], eb0s, sem0)

            @pl.loop(0, nch, step=2)
            def _(c, ce_p=ce_p, eb0s=eb0s, eb1s=eb1s):
                c0 = (startc + c) * ce_p
                pltpu.async_copy(pe_hbm.at[pl.ds(c0 + ce_p, ce_p)], eb1s, sem1)
                pltpu.make_async_copy(
                    pe_hbm.at[pl.ds(c0, ce_p)], eb0s, sem0
                ).wait()
                consume(eb0s)

                @pl.when(c + 2 < nch)
                def _():
                    pltpu.async_copy(
                        pe_hbm.at[pl.ds(c0 + 2 * ce_p, ce_p)], eb0s, sem0
                    )

                pltpu.make_async_copy(
                    pe_hbm.at[pl.ds(c0 + ce_p, ce_p)], eb1s, sem1
                ).wait()
                consume(eb1s)

            _seg_scatter(acc_v, out_hbm, outrow, out_rows, nb, bn, bnp, sem0)

    return agg


_HIGHEST = jax.lax.Precision.HIGHEST


def _mm1(x, w1, bn):
    n, k_dim = x.shape
    h_dim = w1.shape[1]

    def body(x_ref, w_ref, o_ref):
        o_ref[...] = lax.dot_general(
            x_ref[...],
            w_ref[...],
            (((1,), (0,)), ((), ())),
            precision=jax.lax.Precision.DEFAULT,
            preferred_element_type=jnp.float32,
        )

    return pl.pallas_call(
        body,
        grid=(n // bn,),
        in_specs=[
            pl.BlockSpec((bn, k_dim), lambda i: (i, 0)),
            pl.BlockSpec((k_dim, h_dim), lambda i: (0, 0)),
        ],
        out_specs=pl.BlockSpec((bn, h_dim), lambda i: (i, 0)),
        out_shape=jax.ShapeDtypeStruct((n, h_dim), jnp.float32),
    )(x, w1)


def _scale(hist_seg, xw1, bn, bnp):
    n, h_dim = xw1.shape
    nb = n // bn

    def body(hist_ref, xw_ref, y_ref, dinv_ref, yseg_ref):
        hp = hist_ref[0, :, :bn].astype(jnp.float32)
        deg = jnp.sum(hp.T, axis=1, keepdims=True)
        dinv = lax.rsqrt(deg + 1.0)
        dinv_ref[...] = dinv
        y = xw_ref[...] * dinv
        y_ref[...] = y
        yseg_ref[0, :, :bn] = y.T
        yseg_ref[0, :, bn:] = jnp.zeros((h_dim, bnp - bn), jnp.float32)

    return pl.pallas_call(
        body,
        grid=(nb,),
        in_specs=[
            pl.BlockSpec((1, TILES, bnp), lambda i: (i, 0, 0)),
            pl.BlockSpec((bn, h_dim), lambda i: (i, 0)),
        ],
        out_specs=[
            pl.BlockSpec((bn, h_dim), lambda i: (i, 0)),
            pl.BlockSpec((bn, 1), lambda i: (i, 0)),
            pl.BlockSpec((1, h_dim, bnp), lambda i: (i, 0, 0)),
        ],
        out_shape=[
            jax.ShapeDtypeStruct((n, h_dim), jnp.float32),
            jax.ShapeDtypeStruct((n, 1), jnp.float32),
            jax.ShapeDtypeStruct((nb, h_dim, bnp), jnp.float32),
        ],
    )(hist_seg, xw1)


def _layer2(agg1_seg, y1, dinv, b1_row, w2, bn, bnp):
    n, h_dim = y1.shape
    c_dim = w2.shape[1]
    nb = n // bn

    def body(agg_ref, y1_ref, dinv_ref, b1_ref, w2_ref, y2_ref, y2seg_ref):
        agg1 = agg_ref[0, :, :bn].T
        dinv_blk = dinv_ref[...]
        h = jnp.maximum(dinv_blk * (agg1 + y1_ref[...]) + b1_ref[...], 0.0)
        y2 = (
            lax.dot_general(
                h,
                w2_ref[...],
                (((1,), (0,)), ((), ())),
                precision=_HIGHEST,
                preferred_element_type=jnp.float32,
            )
            * dinv_blk
        )
        y2_ref[...] = y2
        y2seg_ref[0, :, :bn] = y2.T
        y2seg_ref[0, :, bn:] = jnp.zeros((c_dim, bnp - bn), jnp.float32)

    return pl.pallas_call(
        body,
        grid=(nb,),
        in_specs=[
            pl.BlockSpec((1, h_dim, bnp), lambda i: (i, 0, 0)),
            pl.BlockSpec((bn, h_dim), lambda i: (i, 0)),
            pl.BlockSpec((bn, 1), lambda i: (i, 0)),
            pl.BlockSpec((1, h_dim), lambda i: (0, 0)),
            pl.BlockSpec((h_dim, c_dim), lambda i: (0, 0)),
        ],
        out_specs=[
            pl.BlockSpec((bn, c_dim), lambda i: (i, 0)),
            pl.BlockSpec((1, c_dim, bnp), lambda i: (i, 0, 0)),
        ],
        out_shape=[
            jax.ShapeDtypeStruct((n, c_dim), jnp.float32),
            jax.ShapeDtypeStruct((nb, c_dim, bnp), jnp.float32),
        ],
    )(agg1_seg, y1, dinv, b1_row, w2)


def _final(agg2_seg, y2, dinv, b2_row, bn, bnp):
    n, c_dim = y2.shape
    nb = n // bn

    def body(agg_ref, y2_ref, dinv_ref, b2_ref, o_ref):
        a = agg_ref[0, :, :bn].T
        tail = a[:, 32:40] + a[:, 40:48] + a[:, 48:56] + a[:, 56:64]
        agg_full = jnp.concatenate([a[:, :32], tail], axis=1)
        o_ref[...] = dinv_ref[...] * (agg_full + y2_ref[...]) + b2_ref[...]

    return pl.pallas_call(
        body,
        grid=(nb,),
        in_specs=[
            pl.BlockSpec((1, 64, bnp), lambda i: (i, 0, 0)),
            pl.BlockSpec((bn, c_dim), lambda i: (i, 0)),
            pl.BlockSpec((bn, 1), lambda i: (i, 0)),
            pl.BlockSpec((1, c_dim), lambda i: (0, 0)),
        ],
        out_specs=pl.BlockSpec((bn, c_dim), lambda i: (i, 0)),
        out_shape=jax.ShapeDtypeStruct((n, c_dim), jnp.float32),
    )(agg2_seg, y2, dinv, b2_row)


def kernel(x, edge_index, W1, b1, W2, b2):
    n, f_in = x.shape
    h_dim = W1.shape[1]
    c_dim = W2.shape[1]
    e = edge_index.shape[1]

    src = edge_index[0].astype(jnp.int32)
    dst = edge_index[1].astype(jnp.int32)
    pe = jnp.bitwise_or(src, jnp.left_shift(dst, 16))

    group = 2 * CE
    ep = ((e + group - 1) // group) * group
    if ep != e:
        sent = jnp.full((ep - e,), n, jnp.int32)
        pe = jnp.concatenate([pe, jnp.bitwise_or(sent, jnp.left_shift(sent, 16))])
    np_ = ((n + LANES + STEP - 1) // STEP) * STEP
    ce = CE
    assert 2 * np_ + 2 * ce <= 131000 and ep % TILES == 0 and (ep // TILES) % STEP == 0

    bn = 2000 if n % 2000 == 0 else max(
        b for b in (1000, 500, 250, 200, 100, 50, 40, 25, 8) if n % b == 0
    )
    nb = n // bn
    bnp = ((bn + 127) // 128) * 128

    nch_full = ep // ce
    ce2 = 2048
    nch_q = ep // (4 * ce2)
    assert ep % (8 * ce2) == 0

    l1_passes = [
        ((lambda wid, p=p: (wid + TILES * p, 0, nch_full, wid + TILES * p)), ce)
        for p in range(h_dim // TILES)
    ]
    l2_passes = [
        ((lambda wid: (wid, 0, nch_full, wid)), ce),
        (
            (lambda wid: (
                32 + (wid >> 2),
                (wid & 3) * nch_q,
                nch_q,
                32 + 8 * (wid & 3) + (wid >> 2),
            )),
            ce2,
        ),
    ]

    hist_seg = _make_hist(n, np_, ep, nb, bn, bnp)(pe)
    xw1 = _mm1(x, W1, bn)
    y1, dinv, y1_seg = _scale(hist_seg.reshape(nb, TILES, bnp), xw1, bn, bnp)
    agg1_seg = _make_agg(
        h_dim, h_dim, n, np_, ep, ce, nb, bn, bnp, l1_passes
    )(y1_seg.reshape(-1), pe)
    y2, y2_seg = _layer2(
        agg1_seg.reshape(nb, h_dim, bnp), y1, dinv, b1.reshape(1, h_dim), W2,
        bn, bnp,
    )
    agg2_seg = _make_agg(
        64, c_dim, n, np_, ep, ce, nb, bn, bnp, l2_passes
    )(y2_seg.reshape(-1), pe)
    return _final(
        agg2_seg.reshape(nb, 64, bnp), y2, dinv, b2.reshape(1, c_dim), bn, bnp
    )


# speedup vs baseline: 23.0637x; 1.0234x over previous
import dataclasses
import functools

import jax
import jax.numpy as jnp
from jax import lax
from jax.experimental import pallas as pl
from jax.experimental.pallas import tpu as pltpu
from jax.experimental.pallas import tpu_sc as plsc

LANES = 16
TILES = 32
UNROLL = 8
STEP = UNROLL * LANES
CE = 8192


def _sc_mesh():
    return plsc.VectorSubcoreMesh(
        core_axis_name="c", subcore_axis_name="s", num_cores=2, num_subcores=16
    )


def _sc_params():
    cp = pltpu.CompilerParams()
    if "needs_layout_passes" in pltpu.CompilerParams.__dataclass_fields__:
        cp = dataclasses.replace(cp, needs_layout_passes=False)
    return cp


def _seg_gather(hbm, col_v, feat, f_dim, nb, bn, bnp, sem):
    for b in range(nb):
        pltpu.async_copy(
            hbm.at[pl.ds((b * f_dim + feat) * bnp, bn)],
            col_v.at[pl.ds(b * bn, bn)],
            sem,
        )
    for b in range(nb):
        pltpu.make_async_copy(
            hbm.at[pl.ds((b * f_dim + feat) * bnp, bn)],
            col_v.at[pl.ds(b * bn, bn)],
            sem,
        ).wait()


def _seg_scatter(col_v, hbm, feat, f_dim, nb, bn, bnp, sem):
    for b in range(nb):
        pltpu.async_copy(
            col_v.at[pl.ds(b * bn, bn)],
            hbm.at[pl.ds((b * f_dim + feat) * bnp, bn)],
            sem,
        )
    for b in range(nb):
        pltpu.make_async_copy(
            col_v.at[pl.ds(b * bn, bn)],
            hbm.at[pl.ds((b * f_dim + feat) * bnp, bn)],
            sem,
        ).wait()


def _make_hist(n: int, np_: int, ep: int, nb: int, bn: int, bnp: int):
    et = ep // TILES

    @functools.partial(
        pl.kernel,
        out_type=jax.ShapeDtypeStruct((nb * TILES * bnp,), jnp.int32),
        mesh=_sc_mesh(),
        scratch_types=[
            pltpu.VMEM((np_,), jnp.int32),
            pltpu.VMEM((et,), jnp.int32),
            pltpu.SemaphoreType.DMA,
        ],
        compiler_params=_sc_params(),
    )
    def hist(pe_hbm, out_hbm, hist_v, ebuf, sem):
        wid = lax.axis_index("s") * 2 + lax.axis_index("c")
        zeros = jnp.zeros((LANES,), jnp.int32)
        ones = jnp.ones((LANES,), jnp.int32)

        @pl.loop(0, np_, step=STEP)
        def _(i):
            for u in range(UNROLL):
                hist_v[pl.ds(i + u * LANES, LANES)] = zeros

        pltpu.sync_copy(pe_hbm.at[pl.ds(wid * et, et)], ebuf)

        @plsc.parallel_loop(0, et, LANES, unroll=UNROLL)
        def _(i):
            pe = ebuf[pl.ds(i, LANES)]
            d = (pe >> 16) & 0xFFFF
            plsc.addupdate_scatter(hist_v, [d], ones)

        _seg_scatter(hist_v, out_hbm, wid, TILES, nb, bn, bnp, sem)

    return hist


def _make_agg(
    out_rows: int,
    in_feats: int,
    n: int,
    np_: int,
    ep: int,
    ce: int,
    nb: int,
    bn: int,
    bnp: int,
    passes,
):
    @functools.partial(
        pl.kernel,
        out_type=jax.ShapeDtypeStruct((nb * out_rows * bnp,), jnp.float32),
        mesh=_sc_mesh(),
        scratch_types=[
            pltpu.VMEM((np_,), jnp.float32),
            pltpu.VMEM((np_,), jnp.float32),
            pltpu.VMEM((ce,), jnp.int32),
            pltpu.VMEM((ce,), jnp.int32),
            pltpu.SemaphoreType.DMA,
            pltpu.SemaphoreType.DMA,
        ],
        compiler_params=_sc_params(),
    )
    def agg(y_hbm, pe_hbm, out_hbm, col_v, acc_v, eb0, eb1, sem0, sem1):
        wid = lax.axis_index("s") * 2 + lax.axis_index("c")
        zeros = jnp.zeros((LANES,), jnp.float32)

        for job, ce_p in passes:
            assert ce_p <= ce

            def consume(ebuf, ce_p=ce_p):
                @plsc.parallel_loop(0, ce_p, LANES, unroll=UNROLL)
                def _(i):
                    pe = ebuf[pl.ds(i, LANES)]
                    s = pe & 0xFFFF
                    d = (pe >> 16) & 0xFFFF
                    v = plsc.load_gather(col_v, [s])
                    plsc.addupdate_scatter(acc_v, [d], v)

            feat, startc, nch, outrow = job(wid)
            assert nch % 2 == 0 and nch * ce_p <= ep

            _seg_gather(y_hbm, col_v, feat, in_feats, nb, bn, bnp, sem0)

            @pl.loop(n, np_, step=LANES)
            def _(i):
                col_v[pl.ds(i, LANES)] = zeros

            @plsc.parallel_loop(0, np_, LANES, unroll=UNROLL)
            def _(i):
                acc_v[pl.ds(i, LANES)] = zeros

            eb0s, eb1s = eb0.at[pl.ds(0, ce_p)], eb1.at[pl.ds(0, ce_p)]
            pltpu.async_copy(pe_hbm.at[pl.ds(startc * ce_p, ce_p)], eb0s, sem0)

            @pl.loop(0, nch, step=2)
            def _(c, ce_p=ce_p, eb0s=eb0s, eb1s=eb1s):
                c0 = (startc + c) * ce_p
                pltpu.async_copy(pe_hbm.at[pl.ds(c0 + ce_p, ce_p)], eb1s, sem1)
                pltpu.make_async_copy(
                    pe_hbm.at[pl.ds(c0, ce_p)], eb0s, sem0
                ).wait()
                consume(eb0s)

                @pl.when(c + 2 < nch)
                def _():
                    pltpu.async_copy(
                        pe_hbm.at[pl.ds(c0 + 2 * ce_p, ce_p)], eb0s, sem0
                    )

                pltpu.make_async_copy(
                    pe_hbm.at[pl.ds(c0 + ce_p, ce_p)], eb1s, sem1
                ).wait()
                consume(eb1s)

            _seg_scatter(acc_v, out_hbm, outrow, out_rows, nb, bn, bnp, sem0)

    return agg


_HIGHEST = jax.lax.Precision.HIGHEST


def _mm1_scale(x, w1, hist_seg, bn, bnp):
    n, k_dim = x.shape
    h_dim = w1.shape[1]
    nb = n // bn

    def body(x_ref, w_ref, hist_ref, y_ref, dinv_ref, yseg_ref):
        xw = lax.dot_general(
            x_ref[...],
            w_ref[...],
            (((1,), (0,)), ((), ())),
            precision=jax.lax.Precision.DEFAULT,
            preferred_element_type=jnp.float32,
        )
        hp = hist_ref[0, :, :bn].astype(jnp.float32)
        deg = jnp.sum(hp.T, axis=1, keepdims=True)
        dinv = lax.rsqrt(deg + 1.0)
        dinv_ref[...] = dinv
        y = xw * dinv
        y_ref[...] = y
        yseg_ref[0, :, :bn] = y.T
        yseg_ref[0, :, bn:] = jnp.zeros((h_dim, bnp - bn), jnp.float32)

    return pl.pallas_call(
        body,
        grid=(nb,),
        in_specs=[
            pl.BlockSpec((bn, k_dim), lambda i: (i, 0)),
            pl.BlockSpec((k_dim, h_dim), lambda i: (0, 0)),
            pl.BlockSpec((1, TILES, bnp), lambda i: (i, 0, 0)),
        ],
        out_specs=[
            pl.BlockSpec((bn, h_dim), lambda i: (i, 0)),
            pl.BlockSpec((bn, 1), lambda i: (i, 0)),
            pl.BlockSpec((1, h_dim, bnp), lambda i: (i, 0, 0)),
        ],
        out_shape=[
            jax.ShapeDtypeStruct((n, h_dim), jnp.float32),
            jax.ShapeDtypeStruct((n, 1), jnp.float32),
            jax.ShapeDtypeStruct((nb, h_dim, bnp), jnp.float32),
        ],
    )(x, w1, hist_seg)


def _layer2(agg1_seg, y1, dinv, b1_row, w2, bn, bnp):
    n, h_dim = y1.shape
    c_dim = w2.shape[1]
    nb = n // bn

    def body(agg_ref, y1_ref, dinv_ref, b1_ref, w2_ref, y2_ref, y2seg_ref):
        agg1 = agg_ref[0, :, :bn].T
        dinv_blk = dinv_ref[...]
        h = jnp.maximum(dinv_blk * (agg1 + y1_ref[...]) + b1_ref[...], 0.0)
        y2 = (
            lax.dot_general(
                h,
                w2_ref[...],
                (((1,), (0,)), ((), ())),
                precision=_HIGHEST,
                preferred_element_type=jnp.float32,
            )
            * dinv_blk
        )
        y2_ref[...] = y2
        y2seg_ref[0, :, :bn] = y2.T
        y2seg_ref[0, :, bn:] = jnp.zeros((c_dim, bnp - bn), jnp.float32)

    return pl.pallas_call(
        body,
        grid=(nb,),
        in_specs=[
            pl.BlockSpec((1, h_dim, bnp), lambda i: (i, 0, 0)),
            pl.BlockSpec((bn, h_dim), lambda i: (i, 0)),
            pl.BlockSpec((bn, 1), lambda i: (i, 0)),
            pl.BlockSpec((1, h_dim), lambda i: (0, 0)),
            pl.BlockSpec((h_dim, c_dim), lambda i: (0, 0)),
        ],
        out_specs=[
            pl.BlockSpec((bn, c_dim), lambda i: (i, 0)),
            pl.BlockSpec((1, c_dim, bnp), lambda i: (i, 0, 0)),
        ],
        out_shape=[
            jax.ShapeDtypeStruct((n, c_dim), jnp.float32),
            jax.ShapeDtypeStruct((nb, c_dim, bnp), jnp.float32),
        ],
    )(agg1_seg, y1, dinv, b1_row, w2)


def _final(agg2_seg, y2, dinv, b2_row, bn, bnp):
    n, c_dim = y2.shape
    nb = n // bn

    def body(agg_ref, y2_ref, dinv_ref, b2_ref, o_ref):
        a = agg_ref[0, :, :bn].T
        tail = a[:, 32:40] + a[:, 40:48] + a[:, 48:56] + a[:, 56:64]
        agg_full = jnp.concatenate([a[:, :32], tail], axis=1)
        o_ref[...] = dinv_ref[...] * (agg_full + y2_ref[...]) + b2_ref[...]

    return pl.pallas_call(
        body,
        grid=(nb,),
        in_specs=[
            pl.BlockSpec((1, 64, bnp), lambda i: (i, 0, 0)),
            pl.BlockSpec((bn, c_dim), lambda i: (i, 0)),
            pl.BlockSpec((bn, 1), lambda i: (i, 0)),
            pl.BlockSpec((1, c_dim), lambda i: (0, 0)),
        ],
        out_specs=pl.BlockSpec((bn, c_dim), lambda i: (i, 0)),
        out_shape=jax.ShapeDtypeStruct((n, c_dim), jnp.float32),
    )(agg2_seg, y2, dinv, b2_row)


def kernel(x, edge_index, W1, b1, W2, b2):
    n, f_in = x.shape
    h_dim = W1.shape[1]
    c_dim = W2.shape[1]
    e = edge_index.shape[1]

    src = edge_index[0].astype(jnp.int32)
    dst = edge_index[1].astype(jnp.int32)
    pe = jnp.bitwise_or(src, jnp.left_shift(dst, 16))

    group = 2 * CE
    ep = ((e + group - 1) // group) * group
    if ep != e:
        sent = jnp.full((ep - e,), n, jnp.int32)
        pe = jnp.concatenate([pe, jnp.bitwise_or(sent, jnp.left_shift(sent, 16))])
    np_ = ((n + LANES + STEP - 1) // STEP) * STEP
    ce = CE
    assert 2 * np_ + 2 * ce <= 131000 and ep % TILES == 0 and (ep // TILES) % STEP == 0

    bn = 2000 if n % 2000 == 0 else max(
        b for b in (1000, 500, 250, 200, 100, 50, 40, 25, 8) if n % b == 0
    )
    nb = n // bn
    bnp = ((bn + 127) // 128) * 128

    nch_full = ep // ce
    ce2 = 2048
    nch_q = ep // (4 * ce2)
    assert ep % (8 * ce2) == 0

    l1_passes = [
        ((lambda wid, p=p: (wid + TILES * p, 0, nch_full, wid + TILES * p)), ce)
        for p in range(h_dim // TILES)
    ]
    l2_passes = [
        ((lambda wid: (wid, 0, nch_full, wid)), ce),
        (
            (lambda wid: (
                32 + (wid >> 2),
                (wid & 3) * nch_q,
                nch_q,
                32 + 8 * (wid & 3) + (wid >> 2),
            )),
            ce2,
        ),
    ]

    hist_seg = _make_hist(n, np_, ep, nb, bn, bnp)(pe)
    y1, dinv, y1_seg = _mm1_scale(
        x, W1, hist_seg.reshape(nb, TILES, bnp), bn, bnp
    )
    agg1_seg = _make_agg(
        h_dim, h_dim, n, np_, ep, ce, nb, bn, bnp, l1_passes
    )(y1_seg.reshape(-1), pe)
    y2, y2_seg = _layer2(
        agg1_seg.reshape(nb, h_dim, bnp), y1, dinv, b1.reshape(1, h_dim), W2,
        bn, bnp,
    )
    agg2_seg = _make_agg(
        64, c_dim, n, np_, ep, ce, nb, bn, bnp, l2_passes
    )(y2_seg.reshape(-1), pe)
    return _final(
        agg2_seg.reshape(nb, 64, bnp), y2, dinv, b2.reshape(1, c_dim), bn, bnp
    )
